# Initial kernel scaffold; baseline (speedup 1.0000x reference)
import jax
import jax.numpy as jnp
from jax.experimental import pallas as pl


def kernel(h, x, arestas, velocidade, atributos_arestas, pxw1, pxb1, pxw2, pxb2, pew1, peb1, pew2, peb2, phw1, phb1, phw2, phb2, pvw1, pvb1, pvw2, pvb2):
    raise NotImplementedError("write your pallas kernel here")


# speedup vs baseline: 11.7429x; 11.7429x over previous
import functools

import jax
import jax.numpy as jnp
from jax import lax
from jax.experimental import pallas as pl
from jax.experimental.pallas import tpu as pltpu
from jax.experimental.pallas import tpu_sc as plsc

NC = 2
NS = 16
NW = NC * NS

D_TAB = 144
D_AGG = 16


def _round_up(a, b):
    return (a + b - 1) // b * b


def _sc_gather(hxp, rowsN, colsN, NP):
    per_w = NP // NW
    W = 40
    nb = per_w // W
    mesh = plsc.VectorSubcoreMesh(core_axis_name="c", subcore_axis_name="s",
                                  num_cores=NC, num_subcores=NS)

    @functools.partial(
        pl.kernel, mesh=mesh,
        compiler_params=pltpu.CompilerParams(use_tc_tiling_on_sc=False),
        out_type=[jax.ShapeDtypeStruct((NP, D_TAB), jnp.float32),
                  jax.ShapeDtypeStruct((NP, D_TAB), jnp.float32)],
        scratch_types=[pltpu.VMEM((per_w,), jnp.int32),
                       pltpu.VMEM((per_w,), jnp.int32),
                       pltpu.VMEM((per_w, D_TAB), jnp.float32),
                       pltpu.VMEM((per_w, D_TAB), jnp.float32),
                       pltpu.SemaphoreType.DMA],
    )
    def k(tab_hbm, r_hbm, c_hbm, gl_hbm, gc_hbm, idxr, idxc, bl, bc, sem):
        s = lax.axis_index("s")
        c = lax.axis_index("c")
        wid = s * NC + c
        pltpu.sync_copy(r_hbm.at[pl.ds(wid * per_w, per_w)], idxr)
        pltpu.sync_copy(c_hbm.at[pl.ds(wid * per_w, per_w)], idxc)
        cps = []
        for j in range(nb):
            cps.append(pltpu.async_copy(
                tab_hbm.at[idxr.at[pl.ds(j * W, W)]],
                bl.at[pl.ds(j * W, W)], sem))
            cps.append(pltpu.async_copy(
                tab_hbm.at[idxc.at[pl.ds(j * W, W)]],
                bc.at[pl.ds(j * W, W)], sem))
        for cp in cps:
            cp.wait()
        pltpu.sync_copy(bl, gl_hbm.at[pl.ds(wid * per_w, per_w)])
        pltpu.sync_copy(bc, gc_hbm.at[pl.ds(wid * per_w, per_w)])

    return k(hxp, rowsN, colsN)


def _tc_mlp_body(gl_ref, gc_ref, ae_ref, h_ref,
                 wl_ref, wc_ref, wr_ref, we_ref, b1_ref,
                 w2_ref, b2_ref,
                 xw1_ref, xb1_ref, xw2_ref, xb2_ref,
                 vw1_ref, vb1_ref, vw2_ref, vb2_ref,
                 t_ref, pv_ref):
    gl = gl_ref[...]
    gc = gc_ref[...]
    hl = gl[:, :128]
    xl = gl[:, 128:144]
    hc = gc[:, :128]
    xc = gc[:, 128:144]
    dif = xl - xc
    rad = jnp.sum(dif * dif, axis=1, keepdims=True)
    a1 = jnp.tanh(
        jnp.dot(hl, wl_ref[...], preferred_element_type=jnp.float32)
        + jnp.dot(hc, wc_ref[...], preferred_element_type=jnp.float32)
        + jnp.dot(ae_ref[...], we_ref[...], preferred_element_type=jnp.float32)
        + rad * wr_ref[...] + b1_ref[...])
    m = jnp.tanh(jnp.dot(a1, w2_ref[...], preferred_element_type=jnp.float32)
                 + b2_ref[...])
    p1 = jnp.tanh(jnp.dot(m, xw1_ref[...], preferred_element_type=jnp.float32)
                  + xb1_ref[...])
    px = jnp.tanh(jnp.sum(p1 * xw2_ref[...], axis=1, keepdims=True)
                  + xb2_ref[...])
    s_val = px * jnp.sum(dif, axis=1, keepdims=True)
    s_m = jnp.sum(m, axis=1, keepdims=True)
    v1 = jnp.tanh(jnp.dot(h_ref[...], vw1_ref[...],
                          preferred_element_type=jnp.float32) + vb1_ref[...])
    pv = jnp.sum(v1 * vw2_ref[...], axis=1, keepdims=True) + vb2_ref[...]
    col = lax.broadcasted_iota(jnp.int32, t_ref.shape, 1)
    t_ref[...] = jnp.where(col == 0, s_val,
                           jnp.where(col == 1, s_m,
                                     jnp.where(col == 2, 1.0, 0.0)))
    pv_ref[...] = jnp.broadcast_to(pv, pv_ref.shape)


def _tc_mlp(gl, gc, aeP, hP, weights, NP, B=1024):
    grid = (NP // B,)
    row = lambda i: (i, 0)
    fix = lambda i: (0, 0)

    def spec(shape, imap):
        return pl.BlockSpec(shape, imap)

    in_specs = [spec((B, D_TAB), row), spec((B, D_TAB), row),
                spec((B, 16), row), spec((B, 128), row)]
    for wshape in [(128, 128), (128, 128), (1, 128), (16, 128), (1, 128),
                   (128, 128), (1, 128),
                   (128, 128), (1, 128), (1, 128), (1, 1),
                   (128, 128), (1, 128), (1, 128), (1, 1)]:
        in_specs.append(spec(wshape, fix))
    out_specs = [spec((B, D_AGG), row), spec((B, 8), row)]
    return pl.pallas_call(
        _tc_mlp_body,
        grid=grid,
        in_specs=in_specs,
        out_specs=out_specs,
        out_shape=[jax.ShapeDtypeStruct((NP, D_AGG), jnp.float32),
                   jax.ShapeDtypeStruct((NP, 8), jnp.float32)],
    )(gl, gc, aeP, hP, *weights)


def _sc_aggregate(T, rows2d, cols1d, zeros_hbm, NP, EP):
    ep_w = EP // NW
    KR = ep_w // 128
    stripe = NP // NS
    mesh = plsc.VectorSubcoreMesh(core_axis_name="c", subcore_axis_name="s",
                                  num_cores=NC, num_subcores=NS)

    @functools.partial(
        pl.kernel, mesh=mesh,
        compiler_params=pltpu.CompilerParams(use_tc_tiling_on_sc=False),
        out_type=jax.ShapeDtypeStruct((NC, NP, D_AGG), jnp.float32),
        scratch_types=[pltpu.VMEM((KR, 128), jnp.int32),
                       pltpu.VMEM((ep_w,), jnp.int32),
                       pltpu.VMEM((ep_w, D_AGG), jnp.float32),
                       pltpu.VMEM_SHARED((NP, D_AGG), jnp.float32),
                       pltpu.SemaphoreType.DMA],
    )
    def k(t_hbm, r_hbm, c_hbm, z_hbm, out_hbm, idxr, idxc, gbuf, acc, sem):
        s = lax.axis_index("s")
        c = lax.axis_index("c")
        wid = s * NC + c
        pltpu.sync_copy(z_hbm.at[pl.ds(s * stripe, stripe)],
                        acc.at[pl.ds(s * stripe, stripe)])
        pltpu.sync_copy(c_hbm.at[pl.ds(wid * ep_w, ep_w)], idxc)
        pltpu.sync_copy(r_hbm.at[pl.ds(wid * KR, KR)], idxr)
        cps = [pltpu.async_copy(t_hbm.at[idxc.at[pl.ds(j * 128, 128)]],
                                gbuf.at[pl.ds(j * 128, 128)], sem)
               for j in range(KR)]
        for cp in cps:
            cp.wait()
        plsc.subcore_barrier()
        cps = [pltpu.async_copy(gbuf.at[pl.ds(j * 128, 128)],
                                acc.at[idxr.at[j]], sem, add=True)
               for j in range(KR)]
        for cp in cps:
            cp.wait()
        plsc.subcore_barrier()
        pltpu.sync_copy(acc.at[pl.ds(s * stripe, stripe)],
                        out_hbm.at[c].at[pl.ds(s * stripe, stripe)])

    return k(T, rows2d, cols1d, zeros_hbm)


def _tc_final_body(part_ref, h_ref, vel_ref, x_ref, pv_ref,
                   a_ref, u_ref, b1_ref, w2_ref, b2_ref,
                   hn_ref, xn_ref, vo_ref):
    acc = part_ref[0] + part_ref[1]
    summed = acc[:, 0:1]
    smi = acc[:, 1:2]
    cnt = acc[:, 2:3]
    media = jnp.where(cnt > 0, summed / cnt, 0.0)
    pv0 = pv_ref[...][:, 0:1]
    velb = vel_ref[...] * pv0 + media
    xn_ref[...] = x_ref[...] + velb
    vo_ref[...] = velb
    hmid = jnp.tanh(jnp.dot(h_ref[...], a_ref[...],
                            preferred_element_type=jnp.float32)
                    + smi * u_ref[...] + b1_ref[...])
    hn_ref[...] = (jnp.dot(hmid, w2_ref[...],
                           preferred_element_type=jnp.float32) + b2_ref[...])


def _tc_final(part, hP, vel8, x8, pv, weights, NP, B=1024):
    grid = (NP // B,)
    row = lambda i: (i, 0)
    fix = lambda i: (0, 0)
    in_specs = [pl.BlockSpec((NC, B, D_AGG), lambda i: (0, i, 0)),
                pl.BlockSpec((B, 128), row), pl.BlockSpec((B, 8), row),
                pl.BlockSpec((B, 8), row), pl.BlockSpec((B, 8), row),
                pl.BlockSpec((128, 128), fix), pl.BlockSpec((1, 128), fix),
                pl.BlockSpec((1, 128), fix), pl.BlockSpec((128, 128), fix),
                pl.BlockSpec((1, 128), fix)]
    out_specs = [pl.BlockSpec((B, 128), row), pl.BlockSpec((B, 8), row),
                 pl.BlockSpec((B, 8), row)]
    return pl.pallas_call(
        _tc_final_body,
        grid=grid,
        in_specs=in_specs,
        out_specs=out_specs,
        out_shape=[jax.ShapeDtypeStruct((NP, 128), jnp.float32),
                   jax.ShapeDtypeStruct((NP, 8), jnp.float32),
                   jax.ShapeDtypeStruct((NP, 8), jnp.float32)],
    )(part, hP, vel8, x8, pv, *weights)


def kernel(h, x, arestas, velocidade, atributos_arestas,
           pxw1, pxb1, pxw2, pxb2,
           pew1, peb1, pew2, peb2,
           phw1, phb1, phw2, phb2,
           pvw1, pvb1, pvw2, pvb2):
    f32 = jnp.float32
    N, ENT = h.shape
    E = arestas.shape[1]
    NP = _round_up(N, NW * 64)
    EP = _round_up(E, NW * 128)

    rows = arestas[0].astype(jnp.int32)
    cols = arestas[1].astype(jnp.int32)

    xp = jnp.pad(x.astype(f32), ((0, 0), (0, 16 - x.shape[1])))
    hxp = jnp.concatenate([h.astype(f32), xp], axis=1)
    rN = jnp.pad(rows[:N], (0, NP - N))
    cN = jnp.pad(cols[:N], (0, NP - N))
    gl, gc = _sc_gather(hxp, rN, cN, NP)

    aeP = jnp.pad(atributos_arestas[:N].astype(f32), ((0, NP - N), (0, 0)))
    hP = jnp.pad(h.astype(f32), ((0, NP - N), (0, 0)))
    mlp_weights = (
        pew1[:, :ENT].T, pew1[:, ENT:2 * ENT].T,
        pew1[:, 2 * ENT].reshape(1, 128), pew1[:, 2 * ENT + 1:].T,
        peb1.reshape(1, 128),
        pew2.T, peb2.reshape(1, 128),
        pxw1.T, pxb1.reshape(1, 128), pxw2.reshape(1, 128),
        pxb2.reshape(1, 1),
        pvw1.T, pvb1.reshape(1, 128), pvw2.reshape(1, 128),
        pvb2.reshape(1, 1),
    )
    T, pv = _tc_mlp(gl, gc, aeP, hP, mlp_weights, NP)

    rowsP = jnp.concatenate(
        [rows, jnp.full((EP - E,), NP - 1, jnp.int32)]).reshape(EP // 128, 128)
    colsP = jnp.pad(cols, (0, EP - E))
    zer = jnp.zeros((NP, D_AGG), f32)
    part = _sc_aggregate(T, rowsP, colsP, zer, NP, EP)

    vel8 = jnp.pad(velocidade.astype(f32), ((0, NP - N), (0, 8 - 3)))
    x8 = jnp.pad(x.astype(f32), ((0, NP - N), (0, 8 - 3)))
    fin_weights = (phw1[:, :ENT].T, phw1[:, ENT].reshape(1, 128),
                   phb1.reshape(1, 128), phw2.T, phb2.reshape(1, 128))
    hn, xn8, vo8 = _tc_final(part, hP, vel8, x8, pv, fin_weights, NP)

    return (hn[:N], xn8[:N, :3], vo8[:N, :3])


# speedup vs baseline: 11.9952x; 1.0215x over previous
import functools

import jax
import jax.numpy as jnp
from jax import lax
from jax.experimental import pallas as pl
from jax.experimental.pallas import tpu as pltpu
from jax.experimental.pallas import tpu_sc as plsc

NC = 2
NS = 16
NW = NC * NS

D_TAB = 144
D_AGG = 16


def _round_up(a, b):
    return (a + b - 1) // b * b


def _sc_gather(hxp, rowsN, colsN, NP):
    per_w = NP // NW
    W = 64
    nb = per_w // W
    mesh = plsc.VectorSubcoreMesh(core_axis_name="c", subcore_axis_name="s",
                                  num_cores=NC, num_subcores=NS)

    @functools.partial(
        pl.kernel, mesh=mesh,
        compiler_params=pltpu.CompilerParams(use_tc_tiling_on_sc=False),
        out_type=[jax.ShapeDtypeStruct((NP, D_TAB), jnp.float32),
                  jax.ShapeDtypeStruct((NP, D_TAB), jnp.float32)],
        scratch_types=[pltpu.VMEM((per_w,), jnp.int32),
                       pltpu.VMEM((per_w,), jnp.int32),
                       pltpu.VMEM((per_w, D_TAB), jnp.float32),
                       pltpu.VMEM((per_w, D_TAB), jnp.float32),
                       pltpu.SemaphoreType.DMA],
    )
    def k(tab_hbm, r_hbm, c_hbm, gl_hbm, gc_hbm, idxr, idxc, bl, bc, sem):
        s = lax.axis_index("s")
        c = lax.axis_index("c")
        wid = s * NC + c
        pltpu.sync_copy(r_hbm.at[pl.ds(wid * per_w, per_w)], idxr)
        pltpu.sync_copy(c_hbm.at[pl.ds(wid * per_w, per_w)], idxc)
        cps = []
        for j in range(nb):
            cps.append(pltpu.async_copy(
                tab_hbm.at[idxr.at[pl.ds(j * W, W)]],
                bl.at[pl.ds(j * W, W)], sem))
            cps.append(pltpu.async_copy(
                tab_hbm.at[idxc.at[pl.ds(j * W, W)]],
                bc.at[pl.ds(j * W, W)], sem))
        for cp in cps:
            cp.wait()
        pltpu.sync_copy(bl, gl_hbm.at[pl.ds(wid * per_w, per_w)])
        pltpu.sync_copy(bc, gc_hbm.at[pl.ds(wid * per_w, per_w)])

    return k(hxp, rowsN, colsN)


def _tc_mlp_body(gl_ref, gc_ref, ae_ref, h_ref,
                 wl_ref, wc_ref, wr_ref, we_ref, b1_ref,
                 w2_ref, b2_ref,
                 xw1_ref, xb1_ref, xw2_ref, xb2_ref,
                 vw1_ref, vb1_ref, vw2_ref, vb2_ref,
                 t_ref, pv_ref):
    gl = gl_ref[...]
    gc = gc_ref[...]
    hl = gl[:, :128]
    xl = gl[:, 128:144]
    hc = gc[:, :128]
    xc = gc[:, 128:144]
    dif = xl - xc
    rad = jnp.sum(dif * dif, axis=1, keepdims=True)
    a1 = jnp.tanh(
        jnp.dot(hl, wl_ref[...], preferred_element_type=jnp.float32)
        + jnp.dot(hc, wc_ref[...], preferred_element_type=jnp.float32)
        + jnp.dot(ae_ref[...], we_ref[...], preferred_element_type=jnp.float32)
        + rad * wr_ref[...] + b1_ref[...])
    m = jnp.tanh(jnp.dot(a1, w2_ref[...], preferred_element_type=jnp.float32)
                 + b2_ref[...])
    p1 = jnp.tanh(jnp.dot(m, xw1_ref[...], preferred_element_type=jnp.float32)
                  + xb1_ref[...])
    px = jnp.tanh(jnp.sum(p1 * xw2_ref[...], axis=1, keepdims=True)
                  + xb2_ref[...])
    s_val = px * jnp.sum(dif, axis=1, keepdims=True)
    s_m = jnp.sum(m, axis=1, keepdims=True)
    v1 = jnp.tanh(jnp.dot(h_ref[...], vw1_ref[...],
                          preferred_element_type=jnp.float32) + vb1_ref[...])
    pv = jnp.sum(v1 * vw2_ref[...], axis=1, keepdims=True) + vb2_ref[...]
    col = lax.broadcasted_iota(jnp.int32, t_ref.shape, 1)
    t_ref[...] = jnp.where(col == 0, s_val,
                           jnp.where(col == 1, s_m,
                                     jnp.where(col == 2, 1.0, 0.0)))
    pv_ref[...] = jnp.broadcast_to(pv, pv_ref.shape)


def _tc_mlp(gl, gc, aeP, hP, weights, NP, B=1024):
    grid = (NP // B,)
    row = lambda i: (i, 0)
    fix = lambda i: (0, 0)

    def spec(shape, imap):
        return pl.BlockSpec(shape, imap)

    in_specs = [spec((B, D_TAB), row), spec((B, D_TAB), row),
                spec((B, 16), row), spec((B, 128), row)]
    for wshape in [(128, 128), (128, 128), (1, 128), (16, 128), (1, 128),
                   (128, 128), (1, 128),
                   (128, 128), (1, 128), (1, 128), (1, 1),
                   (128, 128), (1, 128), (1, 128), (1, 1)]:
        in_specs.append(spec(wshape, fix))
    out_specs = [spec((B, D_AGG), row), spec((B, 8), row)]
    return pl.pallas_call(
        _tc_mlp_body,
        grid=grid,
        in_specs=in_specs,
        out_specs=out_specs,
        out_shape=[jax.ShapeDtypeStruct((NP, D_AGG), jnp.float32),
                   jax.ShapeDtypeStruct((NP, 8), jnp.float32)],
    )(gl, gc, aeP, hP, *weights)


def _sc_aggregate(T, rows2d, cols1d, zeros_hbm, NP, EP):
    ep_w = EP // NW
    KR = ep_w // 128
    stripe = NP // NS
    mesh = plsc.VectorSubcoreMesh(core_axis_name="c", subcore_axis_name="s",
                                  num_cores=NC, num_subcores=NS)

    @functools.partial(
        pl.kernel, mesh=mesh,
        compiler_params=pltpu.CompilerParams(use_tc_tiling_on_sc=False),
        out_type=jax.ShapeDtypeStruct((NC, NP, D_AGG), jnp.float32),
        scratch_types=[pltpu.VMEM((KR, 128), jnp.int32),
                       pltpu.VMEM((ep_w,), jnp.int32),
                       pltpu.VMEM((ep_w, D_AGG), jnp.float32),
                       pltpu.VMEM_SHARED((NP, D_AGG), jnp.float32),
                       pltpu.SemaphoreType.DMA,
                       pltpu.SemaphoreType.DMA],
    )
    def k(t_hbm, r_hbm, c_hbm, z_hbm, out_hbm, idxr, idxc, gbuf, acc,
          sem_a, sem_b):
        s = lax.axis_index("s")
        c = lax.axis_index("c")
        wid = s * NC + c
        pltpu.sync_copy(z_hbm.at[pl.ds(s * stripe, stripe)],
                        acc.at[pl.ds(s * stripe, stripe)])
        pltpu.sync_copy(c_hbm.at[pl.ds(wid * ep_w, ep_w)], idxc)
        pltpu.sync_copy(r_hbm.at[pl.ds(wid * KR, KR)], idxr)
        half = KR // 2
        sems = (sem_a, sem_b)
        gcps = [[pltpu.async_copy(t_hbm.at[idxc.at[pl.ds(j * 128, 128)]],
                                  gbuf.at[pl.ds(j * 128, 128)], sems[kk])
                 for j in range(kk * half, (kk + 1) * half)]
                for kk in range(2)]
        for cp in gcps[0]:
            cp.wait()
        plsc.subcore_barrier()
        for kk in range(2):
            if kk == 1:
                for cp in gcps[1]:
                    cp.wait()
            cps = [pltpu.async_copy(gbuf.at[pl.ds(j * 128, 128)],
                                    acc.at[idxr.at[j]], sems[kk], add=True)
                   for j in range(kk * half, (kk + 1) * half)]
            for cp in cps:
                cp.wait()
        plsc.subcore_barrier()
        pltpu.sync_copy(acc.at[pl.ds(s * stripe, stripe)],
                        out_hbm.at[c].at[pl.ds(s * stripe, stripe)])

    return k(T, rows2d, cols1d, zeros_hbm)


def _tc_final_body(part_ref, h_ref, vel_ref, x_ref, pv_ref,
                   a_ref, u_ref, b1_ref, w2_ref, b2_ref,
                   hn_ref, xn_ref, vo_ref):
    acc = part_ref[0] + part_ref[1]
    summed = acc[:, 0:1]
    smi = acc[:, 1:2]
    cnt = acc[:, 2:3]
    media = jnp.where(cnt > 0, summed / cnt, 0.0)
    pv0 = pv_ref[...][:, 0:1]
    velb = vel_ref[...] * pv0 + media
    xn_ref[...] = x_ref[...] + velb
    vo_ref[...] = velb
    hmid = jnp.tanh(jnp.dot(h_ref[...], a_ref[...],
                            preferred_element_type=jnp.float32)
                    + smi * u_ref[...] + b1_ref[...])
    hn_ref[...] = (jnp.dot(hmid, w2_ref[...],
                           preferred_element_type=jnp.float32) + b2_ref[...])


def _tc_final(part, hP, vel8, x8, pv, weights, NP, B=1024):
    grid = (NP // B,)
    row = lambda i: (i, 0)
    fix = lambda i: (0, 0)
    in_specs = [pl.BlockSpec((NC, B, D_AGG), lambda i: (0, i, 0)),
                pl.BlockSpec((B, 128), row), pl.BlockSpec((B, 8), row),
                pl.BlockSpec((B, 8), row), pl.BlockSpec((B, 8), row),
                pl.BlockSpec((128, 128), fix), pl.BlockSpec((1, 128), fix),
                pl.BlockSpec((1, 128), fix), pl.BlockSpec((128, 128), fix),
                pl.BlockSpec((1, 128), fix)]
    out_specs = [pl.BlockSpec((B, 128), row), pl.BlockSpec((B, 8), row),
                 pl.BlockSpec((B, 8), row)]
    return pl.pallas_call(
        _tc_final_body,
        grid=grid,
        in_specs=in_specs,
        out_specs=out_specs,
        out_shape=[jax.ShapeDtypeStruct((NP, 128), jnp.float32),
                   jax.ShapeDtypeStruct((NP, 8), jnp.float32),
                   jax.ShapeDtypeStruct((NP, 8), jnp.float32)],
    )(part, hP, vel8, x8, pv, *weights)


def kernel(h, x, arestas, velocidade, atributos_arestas,
           pxw1, pxb1, pxw2, pxb2,
           pew1, peb1, pew2, peb2,
           phw1, phb1, phw2, phb2,
           pvw1, pvb1, pvw2, pvb2):
    f32 = jnp.float32
    N, ENT = h.shape
    E = arestas.shape[1]
    NP = _round_up(N, NW * 64)
    EP = _round_up(E, NW * 128)

    rows = arestas[0].astype(jnp.int32)
    cols = arestas[1].astype(jnp.int32)

    xp = jnp.pad(x.astype(f32), ((0, 0), (0, 16 - x.shape[1])))
    hxp = jnp.concatenate([h.astype(f32), xp], axis=1)
    rN = jnp.pad(rows[:N], (0, NP - N))
    cN = jnp.pad(cols[:N], (0, NP - N))
    gl, gc = _sc_gather(hxp, rN, cN, NP)

    aeP = jnp.pad(atributos_arestas[:N].astype(f32), ((0, NP - N), (0, 0)))
    hP = jnp.pad(h.astype(f32), ((0, NP - N), (0, 0)))
    mlp_weights = (
        pew1[:, :ENT].T, pew1[:, ENT:2 * ENT].T,
        pew1[:, 2 * ENT].reshape(1, 128), pew1[:, 2 * ENT + 1:].T,
        peb1.reshape(1, 128),
        pew2.T, peb2.reshape(1, 128),
        pxw1.T, pxb1.reshape(1, 128), pxw2.reshape(1, 128),
        pxb2.reshape(1, 1),
        pvw1.T, pvb1.reshape(1, 128), pvw2.reshape(1, 128),
        pvb2.reshape(1, 1),
    )
    T, pv = _tc_mlp(gl, gc, aeP, hP, mlp_weights, NP)

    rowsP = jnp.concatenate(
        [rows, jnp.full((EP - E,), NP - 1, jnp.int32)]).reshape(EP // 128, 128)
    colsP = jnp.pad(cols, (0, EP - E))
    zer = jnp.zeros((NP, D_AGG), f32)
    part = _sc_aggregate(T, rowsP, colsP, zer, NP, EP)

    vel8 = jnp.pad(velocidade.astype(f32), ((0, NP - N), (0, 8 - 3)))
    x8 = jnp.pad(x.astype(f32), ((0, NP - N), (0, 8 - 3)))
    fin_weights = (phw1[:, :ENT].T, phw1[:, ENT].reshape(1, 128),
                   phb1.reshape(1, 128), phw2.T, phb2.reshape(1, 128))
    hn, xn8, vo8 = _tc_final(part, hP, vel8, x8, pv, fin_weights, NP)

    return (hn[:N], xn8[:N, :3], vo8[:N, :3])


# speedup vs baseline: 12.7053x; 1.0592x over previous
import functools

import jax
import jax.numpy as jnp
from jax import lax
from jax.experimental import pallas as pl
from jax.experimental.pallas import tpu as pltpu
from jax.experimental.pallas import tpu_sc as plsc

NC = 2
NS = 16
NW = NC * NS

D_TAB = 144
D_AGG = 8


def _round_up(a, b):
    return (a + b - 1) // b * b


def _sc_gather(hxp, rowsN, colsN, NP):
    per_w = NP // NW
    chunks = []
    o = 0
    while o < per_w:
        w = min(128, per_w - o)
        chunks.append((o, w))
        o += w
    mesh = plsc.VectorSubcoreMesh(core_axis_name="c", subcore_axis_name="s",
                                  num_cores=NC, num_subcores=NS)

    @functools.partial(
        pl.kernel, mesh=mesh,
        compiler_params=pltpu.CompilerParams(use_tc_tiling_on_sc=False),
        out_type=[jax.ShapeDtypeStruct((NP, D_TAB), jnp.float32),
                  jax.ShapeDtypeStruct((NP, D_TAB), jnp.float32)],
        scratch_types=[pltpu.VMEM((per_w,), jnp.int32),
                       pltpu.VMEM((per_w,), jnp.int32),
                       pltpu.VMEM((per_w, D_TAB), jnp.float32),
                       pltpu.VMEM((per_w, D_TAB), jnp.float32),
                       pltpu.SemaphoreType.DMA],
    )
    def k(tab_hbm, r_hbm, c_hbm, gl_hbm, gc_hbm, idxr, idxc, bl, bc, sem):
        s = lax.axis_index("s")
        c = lax.axis_index("c")
        wid = s * NC + c
        pltpu.sync_copy(r_hbm.at[pl.ds(wid * per_w, per_w)], idxr)
        pltpu.sync_copy(c_hbm.at[pl.ds(wid * per_w, per_w)], idxc)
        cps = []
        for (o, w) in chunks:
            cps.append(pltpu.async_copy(
                tab_hbm.at[idxr.at[pl.ds(o, w)]], bl.at[pl.ds(o, w)], sem))
            cps.append(pltpu.async_copy(
                tab_hbm.at[idxc.at[pl.ds(o, w)]], bc.at[pl.ds(o, w)], sem))
        for cp in cps:
            cp.wait()
        pltpu.sync_copy(bl, gl_hbm.at[pl.ds(wid * per_w, per_w)])
        pltpu.sync_copy(bc, gc_hbm.at[pl.ds(wid * per_w, per_w)])

    return k(hxp, rowsN, colsN)


def _tc_mlp_body(gl_ref, gc_ref, ae_ref, h_ref,
                 wl_ref, wc_ref, wr_ref, we_ref, b1_ref,
                 w2_ref, b2_ref,
                 xw1_ref, xb1_ref, xw2_ref, xb2_ref,
                 vw1_ref, vb1_ref, vw2_ref, vb2_ref,
                 t_ref, pv_ref):
    gl = gl_ref[...]
    gc = gc_ref[...]
    hl = gl[:, :128]
    xl = gl[:, 128:144]
    hc = gc[:, :128]
    xc = gc[:, 128:144]
    dif = xl - xc
    rad = jnp.sum(dif * dif, axis=1, keepdims=True)
    a1 = jnp.tanh(
        jnp.dot(hl, wl_ref[...], preferred_element_type=jnp.float32)
        + jnp.dot(hc, wc_ref[...], preferred_element_type=jnp.float32)
        + jnp.dot(ae_ref[...], we_ref[...], preferred_element_type=jnp.float32)
        + rad * wr_ref[...] + b1_ref[...])
    m = jnp.tanh(jnp.dot(a1, w2_ref[...], preferred_element_type=jnp.float32)
                 + b2_ref[...])
    p1 = jnp.tanh(jnp.dot(m, xw1_ref[...], preferred_element_type=jnp.float32)
                  + xb1_ref[...])
    px = jnp.tanh(jnp.sum(p1 * xw2_ref[...], axis=1, keepdims=True)
                  + xb2_ref[...])
    s_val = px * jnp.sum(dif, axis=1, keepdims=True)
    s_m = jnp.sum(m, axis=1, keepdims=True)
    v1 = jnp.tanh(jnp.dot(h_ref[...], vw1_ref[...],
                          preferred_element_type=jnp.float32) + vb1_ref[...])
    pv = jnp.sum(v1 * vw2_ref[...], axis=1, keepdims=True) + vb2_ref[...]
    col = lax.broadcasted_iota(jnp.int32, t_ref.shape, 1)
    t_ref[...] = jnp.where(col == 0, s_val,
                           jnp.where(col == 1, s_m,
                                     jnp.where(col == 2, 1.0, 0.0)))
    pv_ref[...] = jnp.broadcast_to(pv, pv_ref.shape)


def _tc_mlp(gl, gc, aeP, hP, weights, NP, B=1024):
    grid = (NP // B,)
    row = lambda i: (i, 0)
    fix = lambda i: (0, 0)

    def spec(shape, imap):
        return pl.BlockSpec(shape, imap)

    in_specs = [spec((B, D_TAB), row), spec((B, D_TAB), row),
                spec((B, 16), row), spec((B, 128), row)]
    for wshape in [(128, 128), (128, 128), (1, 128), (16, 128), (1, 128),
                   (128, 128), (1, 128),
                   (128, 128), (1, 128), (1, 128), (1, 1),
                   (128, 128), (1, 128), (1, 128), (1, 1)]:
        in_specs.append(spec(wshape, fix))
    out_specs = [spec((B, D_AGG), row), spec((B, 8), row)]
    return pl.pallas_call(
        _tc_mlp_body,
        grid=grid,
        in_specs=in_specs,
        out_specs=out_specs,
        out_shape=[jax.ShapeDtypeStruct((NP, D_AGG), jnp.float32),
                   jax.ShapeDtypeStruct((NP, 8), jnp.float32)],
    )(gl, gc, aeP, hP, *weights)


def _sc_aggregate(T, rows2d, cols1d, zeros_hbm, NP, EP):
    ep_w = EP // NW
    KR = ep_w // 128
    stripe = NP // NS
    mesh = plsc.VectorSubcoreMesh(core_axis_name="c", subcore_axis_name="s",
                                  num_cores=NC, num_subcores=NS)

    @functools.partial(
        pl.kernel, mesh=mesh,
        compiler_params=pltpu.CompilerParams(use_tc_tiling_on_sc=False),
        out_type=jax.ShapeDtypeStruct((NC, NP, D_AGG), jnp.float32),
        scratch_types=[pltpu.VMEM((KR, 128), jnp.int32),
                       pltpu.VMEM((ep_w,), jnp.int32),
                       pltpu.VMEM((ep_w, D_AGG), jnp.float32),
                       pltpu.VMEM_SHARED((NP, D_AGG), jnp.float32),
                       pltpu.SemaphoreType.DMA,
                       pltpu.SemaphoreType.DMA],
    )
    def k(t_hbm, r_hbm, c_hbm, z_hbm, out_hbm, idxr, idxc, gbuf, acc,
          sem_a, sem_b):
        s = lax.axis_index("s")
        c = lax.axis_index("c")
        wid = s * NC + c
        pltpu.sync_copy(z_hbm.at[pl.ds(s * stripe, stripe)],
                        acc.at[pl.ds(s * stripe, stripe)])
        pltpu.sync_copy(c_hbm.at[pl.ds(wid * ep_w, ep_w)], idxc)
        pltpu.sync_copy(r_hbm.at[pl.ds(wid * KR, KR)], idxr)
        half = KR // 2
        sems = (sem_a, sem_b)
        gcps = [[pltpu.async_copy(t_hbm.at[idxc.at[pl.ds(j * 128, 128)]],
                                  gbuf.at[pl.ds(j * 128, 128)], sems[kk])
                 for j in range(kk * half, (kk + 1) * half)]
                for kk in range(2)]
        for cp in gcps[0]:
            cp.wait()
        plsc.subcore_barrier()
        for kk in range(2):
            if kk == 1:
                for cp in gcps[1]:
                    cp.wait()
            cps = [pltpu.async_copy(gbuf.at[pl.ds(j * 128, 128)],
                                    acc.at[idxr.at[j]], sems[kk], add=True)
                   for j in range(kk * half, (kk + 1) * half)]
            for cp in cps:
                cp.wait()
        plsc.subcore_barrier()
        pltpu.sync_copy(acc.at[pl.ds(s * stripe, stripe)],
                        out_hbm.at[c].at[pl.ds(s * stripe, stripe)])

    return k(T, rows2d, cols1d, zeros_hbm)


def _tc_final_body(part_ref, h_ref, vel_ref, x_ref, pv_ref,
                   a_ref, u_ref, b1_ref, w2_ref, b2_ref,
                   hn_ref, xn_ref, vo_ref):
    acc = part_ref[0] + part_ref[1]
    summed = acc[:, 0:1]
    smi = acc[:, 1:2]
    cnt = acc[:, 2:3]
    media = jnp.where(cnt > 0, summed / cnt, 0.0)
    pv0 = pv_ref[...][:, 0:1]
    velb = vel_ref[...] * pv0 + media
    xn_ref[...] = x_ref[...] + velb
    vo_ref[...] = velb
    hmid = jnp.tanh(jnp.dot(h_ref[...], a_ref[...],
                            preferred_element_type=jnp.float32)
                    + smi * u_ref[...] + b1_ref[...])
    hn_ref[...] = (jnp.dot(hmid, w2_ref[...],
                           preferred_element_type=jnp.float32) + b2_ref[...])


def _tc_final(part, hP, vel8, x8, pv, weights, NP, B=1024):
    grid = (NP // B,)
    row = lambda i: (i, 0)
    fix = lambda i: (0, 0)
    in_specs = [pl.BlockSpec((NC, B, D_AGG), lambda i: (0, i, 0)),
                pl.BlockSpec((B, 128), row), pl.BlockSpec((B, 8), row),
                pl.BlockSpec((B, 8), row), pl.BlockSpec((B, 8), row),
                pl.BlockSpec((128, 128), fix), pl.BlockSpec((1, 128), fix),
                pl.BlockSpec((1, 128), fix), pl.BlockSpec((128, 128), fix),
                pl.BlockSpec((1, 128), fix)]
    out_specs = [pl.BlockSpec((B, 128), row), pl.BlockSpec((B, 8), row),
                 pl.BlockSpec((B, 8), row)]
    return pl.pallas_call(
        _tc_final_body,
        grid=grid,
        in_specs=in_specs,
        out_specs=out_specs,
        out_shape=[jax.ShapeDtypeStruct((NP, 128), jnp.float32),
                   jax.ShapeDtypeStruct((NP, 8), jnp.float32),
                   jax.ShapeDtypeStruct((NP, 8), jnp.float32)],
    )(part, hP, vel8, x8, pv, *weights)


def kernel(h, x, arestas, velocidade, atributos_arestas,
           pxw1, pxb1, pxw2, pxb2,
           pew1, peb1, pew2, peb2,
           phw1, phb1, phw2, phb2,
           pvw1, pvb1, pvw2, pvb2):
    f32 = jnp.float32
    N, ENT = h.shape
    E = arestas.shape[1]
    NP = _round_up(N, NW * 64)
    EP = _round_up(E, NW * 128)

    rows = arestas[0].astype(jnp.int32)
    cols = arestas[1].astype(jnp.int32)

    xp = jnp.pad(x.astype(f32), ((0, 0), (0, 16 - x.shape[1])))
    hxp = jnp.concatenate([h.astype(f32), xp], axis=1)
    rN = jnp.pad(rows[:N], (0, NP - N))
    cN = jnp.pad(cols[:N], (0, NP - N))
    gl, gc = _sc_gather(hxp, rN, cN, NP)

    aeP = jnp.pad(atributos_arestas[:N].astype(f32), ((0, NP - N), (0, 0)))
    hP = jnp.pad(h.astype(f32), ((0, NP - N), (0, 0)))
    mlp_weights = (
        pew1[:, :ENT].T, pew1[:, ENT:2 * ENT].T,
        pew1[:, 2 * ENT].reshape(1, 128), pew1[:, 2 * ENT + 1:].T,
        peb1.reshape(1, 128),
        pew2.T, peb2.reshape(1, 128),
        pxw1.T, pxb1.reshape(1, 128), pxw2.reshape(1, 128),
        pxb2.reshape(1, 1),
        pvw1.T, pvb1.reshape(1, 128), pvw2.reshape(1, 128),
        pvb2.reshape(1, 1),
    )
    T, pv = _tc_mlp(gl, gc, aeP, hP, mlp_weights, NP)

    rowsP = jnp.concatenate(
        [rows, jnp.full((EP - E,), NP - 1, jnp.int32)]).reshape(EP // 128, 128)
    colsP = jnp.pad(cols, (0, EP - E))
    zer = jnp.zeros((NP, D_AGG), f32)
    part = _sc_aggregate(T, rowsP, colsP, zer, NP, EP)

    vel8 = jnp.pad(velocidade.astype(f32), ((0, NP - N), (0, 8 - 3)))
    x8 = jnp.pad(x.astype(f32), ((0, NP - N), (0, 8 - 3)))
    fin_weights = (phw1[:, :ENT].T, phw1[:, ENT].reshape(1, 128),
                   phb1.reshape(1, 128), phw2.T, phb2.reshape(1, 128))
    hn, xn8, vo8 = _tc_final(part, hP, vel8, x8, pv, fin_weights, NP)

    return (hn[:N], xn8[:N, :3], vo8[:N, :3])


# speedup vs baseline: 13.4443x; 1.0582x over previous
import functools

import jax
import jax.numpy as jnp
from jax import lax
from jax.experimental import pallas as pl
from jax.experimental.pallas import tpu as pltpu
from jax.experimental.pallas import tpu_sc as plsc

NC = 2
NS = 16
NW = NC * NS

D_TAB = 144
D_AGG = 8
D_TBL = 2


def _round_up(a, b):
    return (a + b - 1) // b * b


def _sc_gather(hxp, rowsN, colsN, NP):
    per_w = NP // NW
    chunks = []
    o = 0
    while o < per_w:
        w = min(128, per_w - o)
        chunks.append((o, w))
        o += w
    mesh = plsc.VectorSubcoreMesh(core_axis_name="c", subcore_axis_name="s",
                                  num_cores=NC, num_subcores=NS)

    @functools.partial(
        pl.kernel, mesh=mesh,
        compiler_params=pltpu.CompilerParams(use_tc_tiling_on_sc=False),
        out_type=[jax.ShapeDtypeStruct((NP, D_TAB), jnp.float32),
                  jax.ShapeDtypeStruct((NP, D_TAB), jnp.float32)],
        scratch_types=[pltpu.VMEM((per_w,), jnp.int32),
                       pltpu.VMEM((per_w,), jnp.int32),
                       pltpu.VMEM((per_w, D_TAB), jnp.float32),
                       pltpu.VMEM((per_w, D_TAB), jnp.float32),
                       pltpu.SemaphoreType.DMA],
    )
    def k(tab_hbm, r_hbm, c_hbm, gl_hbm, gc_hbm, idxr, idxc, bl, bc, sem):
        s = lax.axis_index("s")
        c = lax.axis_index("c")
        wid = s * NC + c
        pltpu.sync_copy(r_hbm.at[pl.ds(wid * per_w, per_w)], idxr)
        pltpu.sync_copy(c_hbm.at[pl.ds(wid * per_w, per_w)], idxc)
        cps = []
        for (o, w) in chunks:
            cps.append(pltpu.async_copy(
                tab_hbm.at[idxr.at[pl.ds(o, w)]], bl.at[pl.ds(o, w)], sem))
            cps.append(pltpu.async_copy(
                tab_hbm.at[idxc.at[pl.ds(o, w)]], bc.at[pl.ds(o, w)], sem))
        for cp in cps:
            cp.wait()
        pltpu.sync_copy(bl, gl_hbm.at[pl.ds(wid * per_w, per_w)])
        pltpu.sync_copy(bc, gc_hbm.at[pl.ds(wid * per_w, per_w)])

    return k(hxp, rowsN, colsN)


def _tc_mlp_body(gl_ref, gc_ref, ae_ref, h_ref,
                 wl_ref, wc_ref, wr_ref, we_ref, b1_ref,
                 w2_ref, b2_ref,
                 xw1_ref, xb1_ref, xw2_ref, xb2_ref,
                 vw1_ref, vb1_ref, vw2_ref, vb2_ref,
                 t_ref, pv_ref):
    gl = gl_ref[...]
    gc = gc_ref[...]
    hl = gl[:, :128]
    xl = gl[:, 128:144]
    hc = gc[:, :128]
    xc = gc[:, 128:144]
    dif = xl - xc
    rad = jnp.sum(dif * dif, axis=1, keepdims=True)
    a1 = jnp.tanh(
        jnp.dot(hl, wl_ref[...], preferred_element_type=jnp.float32)
        + jnp.dot(hc, wc_ref[...], preferred_element_type=jnp.float32)
        + jnp.dot(ae_ref[...], we_ref[...], preferred_element_type=jnp.float32)
        + rad * wr_ref[...] + b1_ref[...])
    m = jnp.tanh(jnp.dot(a1, w2_ref[...], preferred_element_type=jnp.float32)
                 + b2_ref[...])
    p1 = jnp.tanh(jnp.dot(m, xw1_ref[...], preferred_element_type=jnp.float32)
                  + xb1_ref[...])
    px = jnp.tanh(jnp.sum(p1 * xw2_ref[...], axis=1, keepdims=True)
                  + xb2_ref[...])
    s_val = px * jnp.sum(dif, axis=1, keepdims=True)
    s_m = jnp.sum(m, axis=1, keepdims=True)
    v1 = jnp.tanh(jnp.dot(h_ref[...], vw1_ref[...],
                          preferred_element_type=jnp.float32) + vb1_ref[...])
    pv = jnp.sum(v1 * vw2_ref[...], axis=1, keepdims=True) + vb2_ref[...]
    col = lax.broadcasted_iota(jnp.int32, t_ref.shape, 1)
    t_ref[...] = jnp.where(col == 0, s_val,
                           jnp.where(col == 1, s_m, 0.0))
    pv_ref[...] = jnp.broadcast_to(pv, pv_ref.shape)


def _tc_mlp(gl, gc, aeP, hP, weights, NP, B=1024):
    grid = (NP // B,)
    row = lambda i: (i, 0)
    fix = lambda i: (0, 0)

    def spec(shape, imap):
        return pl.BlockSpec(shape, imap)

    in_specs = [spec((B, D_TAB), row), spec((B, D_TAB), row),
                spec((B, 16), row), spec((B, 128), row)]
    for wshape in [(128, 128), (128, 128), (1, 128), (16, 128), (1, 128),
                   (128, 128), (1, 128),
                   (128, 128), (1, 128), (1, 128), (1, 1),
                   (128, 128), (1, 128), (1, 128), (1, 1)]:
        in_specs.append(spec(wshape, fix))
    out_specs = [spec((B, D_TBL), row), spec((B, 8), row)]
    return pl.pallas_call(
        _tc_mlp_body,
        grid=grid,
        in_specs=in_specs,
        out_specs=out_specs,
        out_shape=[jax.ShapeDtypeStruct((NP, D_TBL), jnp.float32),
                   jax.ShapeDtypeStruct((NP, 8), jnp.float32)],
    )(gl, gc, aeP, hP, *weights)


def _sc_aggregate(T, rows2d, cols1d, zeros_hbm, NP, EP):
    ep_w = EP // NW
    KR = ep_w // 128
    stripe = NP // NS
    mesh = plsc.VectorSubcoreMesh(core_axis_name="c", subcore_axis_name="s",
                                  num_cores=NC, num_subcores=NS)

    @functools.partial(
        pl.kernel, mesh=mesh,
        compiler_params=pltpu.CompilerParams(use_tc_tiling_on_sc=False,
                                             needs_layout_passes=False),
        out_type=jax.ShapeDtypeStruct((NC, NP, D_AGG), jnp.float32),
        scratch_types=[pltpu.VMEM((KR, 128), jnp.int32),
                       pltpu.VMEM((ep_w,), jnp.int32),
                       pltpu.VMEM((NP * D_TBL,), jnp.float32),
                       pltpu.VMEM((ep_w, D_AGG), jnp.float32),
                       pltpu.VMEM_SHARED((NP, D_AGG), jnp.float32),
                       pltpu.SemaphoreType.DMA],
    )
    def k(t_hbm, r_hbm, c_hbm, z_hbm, out_hbm, idxr, idxc, tv, gbuf, acc,
          sem):
        s = lax.axis_index("s")
        c = lax.axis_index("c")
        wid = s * NC + c
        pltpu.sync_copy(z_hbm.at[pl.ds(s * stripe, stripe)],
                        acc.at[pl.ds(s * stripe, stripe)])
        pltpu.sync_copy(t_hbm, tv)
        pltpu.sync_copy(c_hbm.at[pl.ds(wid * ep_w, ep_w)], idxc)
        pltpu.sync_copy(r_hbm.at[pl.ds(wid * KR, KR)], idxr)
        lane = lax.iota(jnp.int32, 16)
        zero16 = jnp.zeros((16,), jnp.int32)
        one16 = jnp.full((16,), 1, jnp.int32)
        two16 = jnp.full((16,), 2, jnp.int32)
        ones_f = jnp.full((16,), 1.0, jnp.float32)

        def body(i, carry):
            ci2 = idxc[pl.ds(i * 16, 16)] * 2
            sv = plsc.load_gather(tv, [ci2])
            sm = plsc.load_gather(tv, [ci2 + one16])
            e16 = lane + i * 16
            plsc.store_scatter(gbuf, [e16, zero16], sv)
            plsc.store_scatter(gbuf, [e16, one16], sm)
            plsc.store_scatter(gbuf, [e16, two16], ones_f)
            return carry

        lax.fori_loop(0, ep_w // 16, body, 0)
        plsc.subcore_barrier()
        cps = [pltpu.async_copy(gbuf.at[pl.ds(j * 128, 128)],
                                acc.at[idxr.at[j]], sem, add=True)
               for j in range(KR)]
        for cp in cps:
            cp.wait()
        plsc.subcore_barrier()
        pltpu.sync_copy(acc.at[pl.ds(s * stripe, stripe)],
                        out_hbm.at[c].at[pl.ds(s * stripe, stripe)])

    return k(T, rows2d, cols1d, zeros_hbm)


def _tc_final_body(part_ref, h_ref, vel_ref, x_ref, pv_ref,
                   a_ref, u_ref, b1_ref, w2_ref, b2_ref,
                   hn_ref, xn_ref, vo_ref):
    acc = part_ref[0] + part_ref[1]
    summed = acc[:, 0:1]
    smi = acc[:, 1:2]
    cnt = acc[:, 2:3]
    media = jnp.where(cnt > 0, summed / cnt, 0.0)
    pv0 = pv_ref[...][:, 0:1]
    velb = vel_ref[...] * pv0 + media
    xn_ref[...] = x_ref[...] + velb
    vo_ref[...] = velb
    hmid = jnp.tanh(jnp.dot(h_ref[...], a_ref[...],
                            preferred_element_type=jnp.float32)
                    + smi * u_ref[...] + b1_ref[...])
    hn_ref[...] = (jnp.dot(hmid, w2_ref[...],
                           preferred_element_type=jnp.float32) + b2_ref[...])


def _tc_final(part, hP, vel8, x8, pv, weights, NP, B=1024):
    grid = (NP // B,)
    row = lambda i: (i, 0)
    fix = lambda i: (0, 0)
    in_specs = [pl.BlockSpec((NC, B, D_AGG), lambda i: (0, i, 0)),
                pl.BlockSpec((B, 128), row), pl.BlockSpec((B, 8), row),
                pl.BlockSpec((B, 8), row), pl.BlockSpec((B, 8), row),
                pl.BlockSpec((128, 128), fix), pl.BlockSpec((1, 128), fix),
                pl.BlockSpec((1, 128), fix), pl.BlockSpec((128, 128), fix),
                pl.BlockSpec((1, 128), fix)]
    out_specs = [pl.BlockSpec((B, 128), row), pl.BlockSpec((B, 8), row),
                 pl.BlockSpec((B, 8), row)]
    return pl.pallas_call(
        _tc_final_body,
        grid=grid,
        in_specs=in_specs,
        out_specs=out_specs,
        out_shape=[jax.ShapeDtypeStruct((NP, 128), jnp.float32),
                   jax.ShapeDtypeStruct((NP, 8), jnp.float32),
                   jax.ShapeDtypeStruct((NP, 8), jnp.float32)],
    )(part, hP, vel8, x8, pv, *weights)


def kernel(h, x, arestas, velocidade, atributos_arestas,
           pxw1, pxb1, pxw2, pxb2,
           pew1, peb1, pew2, peb2,
           phw1, phb1, phw2, phb2,
           pvw1, pvb1, pvw2, pvb2):
    f32 = jnp.float32
    N, ENT = h.shape
    E = arestas.shape[1]
    NP = _round_up(N, NW * 64)
    EP = _round_up(E, NW * 128)

    rows = arestas[0].astype(jnp.int32)
    cols = arestas[1].astype(jnp.int32)

    xp = jnp.pad(x.astype(f32), ((0, 0), (0, 16 - x.shape[1])))
    hxp = jnp.concatenate([h.astype(f32), xp], axis=1)
    rN = jnp.pad(rows[:N], (0, NP - N))
    cN = jnp.pad(cols[:N], (0, NP - N))
    gl, gc = _sc_gather(hxp, rN, cN, NP)

    aeP = jnp.pad(atributos_arestas[:N].astype(f32), ((0, NP - N), (0, 0)))
    hP = jnp.pad(h.astype(f32), ((0, NP - N), (0, 0)))
    mlp_weights = (
        pew1[:, :ENT].T, pew1[:, ENT:2 * ENT].T,
        pew1[:, 2 * ENT].reshape(1, 128), pew1[:, 2 * ENT + 1:].T,
        peb1.reshape(1, 128),
        pew2.T, peb2.reshape(1, 128),
        pxw1.T, pxb1.reshape(1, 128), pxw2.reshape(1, 128),
        pxb2.reshape(1, 1),
        pvw1.T, pvb1.reshape(1, 128), pvw2.reshape(1, 128),
        pvb2.reshape(1, 1),
    )
    T, pv = _tc_mlp(gl, gc, aeP, hP, mlp_weights, NP)

    rowsP = jnp.concatenate(
        [rows, jnp.full((EP - E,), NP - 1, jnp.int32)]).reshape(EP // 128, 128)
    colsP = jnp.pad(cols, (0, EP - E))
    zer = jnp.zeros((NP, D_AGG), f32)
    part = _sc_aggregate(T.reshape(NP * D_TBL), rowsP, colsP, zer, NP, EP)

    vel8 = jnp.pad(velocidade.astype(f32), ((0, NP - N), (0, 8 - 3)))
    x8 = jnp.pad(x.astype(f32), ((0, NP - N), (0, 8 - 3)))
    fin_weights = (phw1[:, :ENT].T, phw1[:, ENT].reshape(1, 128),
                   phb1.reshape(1, 128), phw2.T, phb2.reshape(1, 128))
    hn, xn8, vo8 = _tc_final(part, hP, vel8, x8, pv, fin_weights, NP)

    return (hn[:N], xn8[:N, :3], vo8[:N, :3])


# speedup vs baseline: 14.9319x; 1.1107x over previous
import functools

import jax
import jax.numpy as jnp
from jax import lax
from jax.experimental import pallas as pl
from jax.experimental.pallas import tpu as pltpu
from jax.experimental.pallas import tpu_sc as plsc

NC = 2
NS = 16
NW = NC * NS

D_TAB = 144
D_AGG = 8
D_TBL = 2


def _round_up(a, b):
    return (a + b - 1) // b * b


def _sc_gather(h, xt, rowsN, colsN, NP):
    per_w = NP // NW
    half = per_w // 2
    per_c = per_w * NS
    n_str = per_c // 1024
    chunks = []
    o = 0
    while o < half:
        w = min(128, half - o)
        chunks.append((o, w))
        o += w
    n4 = xt.shape[0]
    mesh = plsc.VectorSubcoreMesh(core_axis_name="c", subcore_axis_name="s",
                                  num_cores=NC, num_subcores=NS)

    @functools.partial(
        pl.kernel, mesh=mesh,
        compiler_params=pltpu.CompilerParams(needs_layout_passes=False),
        out_type=[jax.ShapeDtypeStruct((NP, 128), jnp.float32),
                  jax.ShapeDtypeStruct((NP, 128), jnp.float32),
                  jax.ShapeDtypeStruct((NP,), jnp.float32),
                  jax.ShapeDtypeStruct((NP,), jnp.float32)],
        scratch_types=[pltpu.VMEM((per_w,), jnp.int32),
                       pltpu.VMEM((per_w,), jnp.int32),
                       pltpu.VMEM((n4,), jnp.float32),
                       pltpu.VMEM((half, 128), jnp.float32),
                       pltpu.VMEM((half, 128), jnp.float32),
                       pltpu.VMEM((per_w,), jnp.float32),
                       pltpu.VMEM((per_w,), jnp.float32),
                       pltpu.VMEM_SHARED((per_c,), jnp.float32),
                       pltpu.VMEM_SHARED((per_c,), jnp.float32),
                       pltpu.SemaphoreType.DMA],
    )
    def k(h_hbm, xt_hbm, r_hbm, c_hbm, hl_hbm, hc_hbm, rad_hbm, sd_hbm,
          idxr, idxc, xv, bl, bc, rsv, sdv, rs_sh, sd_sh, sem):
        s = lax.axis_index("s")
        c = lax.axis_index("c")
        wid = c * NS + s
        base = wid * per_w
        pltpu.sync_copy(r_hbm.at[pl.ds(base, per_w)], idxr)
        pltpu.sync_copy(c_hbm.at[pl.ds(base, per_w)], idxc)
        pltpu.sync_copy(xt_hbm, xv)
        cps = []
        for (o, w) in chunks:
            cps.append(pltpu.async_copy(
                h_hbm.at[idxr.at[pl.ds(o, w)]], bl.at[pl.ds(o, w)], sem))
            cps.append(pltpu.async_copy(
                h_hbm.at[idxc.at[pl.ds(o, w)]], bc.at[pl.ds(o, w)], sem))

        def body(i, carry):
            r4 = idxr[pl.ds(i * 16, 16)] * 4
            c4 = idxc[pl.ds(i * 16, 16)] * 4
            d0 = plsc.load_gather(xv, [r4]) - plsc.load_gather(xv, [c4])
            d1 = (plsc.load_gather(xv, [r4 + 1])
                  - plsc.load_gather(xv, [c4 + 1]))
            d2 = (plsc.load_gather(xv, [r4 + 2])
                  - plsc.load_gather(xv, [c4 + 2]))
            rsv[pl.ds(i * 16, 16)] = d0 * d0 + d1 * d1 + d2 * d2
            sdv[pl.ds(i * 16, 16)] = d0 + d1 + d2
            return carry

        lax.fori_loop(0, per_w // 16, body, 0)
        for cp in cps:
            cp.wait()
        pltpu.sync_copy(bl, hl_hbm.at[pl.ds(base, half)])
        pltpu.sync_copy(bc, hc_hbm.at[pl.ds(base, half)])
        cps = []
        for (o, w) in chunks:
            cps.append(pltpu.async_copy(
                h_hbm.at[idxr.at[pl.ds(half + o, w)]], bl.at[pl.ds(o, w)],
                sem))
            cps.append(pltpu.async_copy(
                h_hbm.at[idxc.at[pl.ds(half + o, w)]], bc.at[pl.ds(o, w)],
                sem))
        for cp in cps:
            cp.wait()
        pltpu.sync_copy(bl, hl_hbm.at[pl.ds(base + half, half)])
        pltpu.sync_copy(bc, hc_hbm.at[pl.ds(base + half, half)])
        pltpu.sync_copy(rsv, rs_sh.at[pl.ds(s * per_w, per_w)])
        pltpu.sync_copy(sdv, sd_sh.at[pl.ds(s * per_w, per_w)])
        plsc.subcore_barrier()

        @pl.when(s < n_str)
        def _():
            pltpu.sync_copy(rs_sh.at[pl.ds(s * 1024, 1024)],
                            rad_hbm.at[pl.ds(c * per_c + s * 1024, 1024)])
            pltpu.sync_copy(sd_sh.at[pl.ds(s * 1024, 1024)],
                            sd_hbm.at[pl.ds(c * per_c + s * 1024, 1024)])

    return k(h, xt, rowsN, colsN)


def _tc_mlp_body(hl_ref, hc_ref, ae_ref, h_ref,
                 wl_ref, wc_ref, we_ref, b1_ref,
                 w2_ref, b2_ref,
                 xw1_ref, xb1_ref, xw2_ref, xb2_ref,
                 vw1_ref, vb1_ref, vw2_ref, vb2_ref,
                 t_ref, pv_ref):
    a1 = jnp.tanh(
        jnp.dot(hl_ref[...], wl_ref[...], preferred_element_type=jnp.float32)
        + jnp.dot(hc_ref[...], wc_ref[...],
                  preferred_element_type=jnp.float32)
        + jnp.dot(ae_ref[...], we_ref[...], preferred_element_type=jnp.float32)
        + b1_ref[...])
    m = jnp.tanh(jnp.dot(a1, w2_ref[...], preferred_element_type=jnp.float32)
                 + b2_ref[...])
    p1 = jnp.tanh(jnp.dot(m, xw1_ref[...], preferred_element_type=jnp.float32)
                  + xb1_ref[...])
    px = jnp.tanh(jnp.sum(p1 * xw2_ref[...], axis=1, keepdims=True)
                  + xb2_ref[...])
    s_m = jnp.sum(m, axis=1, keepdims=True)
    v1 = jnp.tanh(jnp.dot(h_ref[...], vw1_ref[...],
                          preferred_element_type=jnp.float32) + vb1_ref[...])
    pv = jnp.sum(v1 * vw2_ref[...], axis=1, keepdims=True) + vb2_ref[...]
    col = lax.broadcasted_iota(jnp.int32, t_ref.shape, 1)
    t_ref[...] = jnp.where(col == 0, px,
                           jnp.where(col == 1, s_m, 0.0))
    pv_ref[...] = jnp.broadcast_to(pv, pv_ref.shape)


def _tc_mlp(hl, hc, aeplus, hP, weights, NP, B=1024):
    grid = (NP // B,)
    row = lambda i: (i, 0)
    fix = lambda i: (0, 0)

    def spec(shape, imap):
        return pl.BlockSpec(shape, imap)

    in_specs = [spec((B, 128), row), spec((B, 128), row),
                spec((B, 24), row), spec((B, 128), row)]
    for wshape in [(128, 128), (128, 128), (24, 128), (1, 128),
                   (128, 128), (1, 128),
                   (128, 128), (1, 128), (1, 128), (1, 1),
                   (128, 128), (1, 128), (1, 128), (1, 1)]:
        in_specs.append(spec(wshape, fix))
    out_specs = [spec((B, 8), row), spec((B, 8), row)]
    return pl.pallas_call(
        _tc_mlp_body,
        grid=grid,
        in_specs=in_specs,
        out_specs=out_specs,
        out_shape=[jax.ShapeDtypeStruct((NP, 8), jnp.float32),
                   jax.ShapeDtypeStruct((NP, 8), jnp.float32)],
    )(hl, hc, aeplus, hP, *weights)


def _sc_aggregate(px2d, sm2d, sd2d, rows2d, cols1d, zeros_hbm, NP, EP):
    ep_w = EP // NW
    KR = ep_w // 128
    stripe = NP // NS
    mesh = plsc.VectorSubcoreMesh(core_axis_name="c", subcore_axis_name="s",
                                  num_cores=NC, num_subcores=NS)

    @functools.partial(
        pl.kernel, mesh=mesh,
        compiler_params=pltpu.CompilerParams(use_tc_tiling_on_sc=False,
                                             needs_layout_passes=False),
        out_type=jax.ShapeDtypeStruct((NC, NP, D_AGG), jnp.float32),
        scratch_types=[pltpu.VMEM((KR, 128), jnp.int32),
                       pltpu.VMEM((ep_w,), jnp.int32),
                       pltpu.VMEM((NP // 128, 128), jnp.float32),
                       pltpu.VMEM((NP // 128, 128), jnp.float32),
                       pltpu.VMEM((NP // 128, 128), jnp.float32),
                       pltpu.VMEM((ep_w, D_AGG), jnp.float32),
                       pltpu.VMEM_SHARED((NP, D_AGG), jnp.float32),
                       pltpu.SemaphoreType.DMA],
    )
    def k(px_hbm, sm_hbm, sd_hbm, r_hbm, c_hbm, z_hbm, out_hbm, idxr, idxc,
          tvs, tvm, tvd, gbuf, acc, sem):
        s = lax.axis_index("s")
        c = lax.axis_index("c")
        wid = s * NC + c
        pltpu.sync_copy(z_hbm.at[pl.ds(s * stripe, stripe)],
                        acc.at[pl.ds(s * stripe, stripe)])
        pltpu.sync_copy(px_hbm, tvs)
        pltpu.sync_copy(sm_hbm, tvm)
        pltpu.sync_copy(sd_hbm, tvd)
        pltpu.sync_copy(c_hbm.at[pl.ds(wid * ep_w, ep_w)], idxc)
        pltpu.sync_copy(r_hbm.at[pl.ds(wid * KR, KR)], idxr)
        lane = lax.iota(jnp.int32, 16)
        zero16 = jnp.zeros((16,), jnp.int32)
        one16 = jnp.full((16,), 1, jnp.int32)
        two16 = jnp.full((16,), 2, jnp.int32)
        ones_f = jnp.full((16,), 1.0, jnp.float32)

        def body(i, carry):
            ci = idxc[pl.ds(i * 16, 16)]
            tr = lax.shift_right_logical(ci, 7)
            tc_ = jnp.bitwise_and(ci, 127)
            sv = (plsc.load_gather(tvs, [tr, tc_])
                  * plsc.load_gather(tvd, [tr, tc_]))
            sm = plsc.load_gather(tvm, [tr, tc_])
            e16 = lane + i * 16
            plsc.store_scatter(gbuf, [e16, zero16], sv)
            plsc.store_scatter(gbuf, [e16, one16], sm)
            plsc.store_scatter(gbuf, [e16, two16], ones_f)
            return carry

        lax.fori_loop(0, ep_w // 16, body, 0)
        plsc.subcore_barrier()
        cps = [pltpu.async_copy(gbuf.at[pl.ds(j * 128, 128)],
                                acc.at[idxr.at[j]], sem, add=True)
               for j in range(KR)]
        for cp in cps:
            cp.wait()
        plsc.subcore_barrier()
        pltpu.sync_copy(acc.at[pl.ds(s * stripe, stripe)],
                        out_hbm.at[c].at[pl.ds(s * stripe, stripe)])

    return k(px2d, sm2d, sd2d, rows2d, cols1d, zeros_hbm)


def _tc_final_body(part_ref, h_ref, vel_ref, x_ref, pv_ref,
                   a_ref, u_ref, b1_ref, w2_ref, b2_ref,
                   hn_ref, xn_ref, vo_ref):
    acc = part_ref[0] + part_ref[1]
    summed = acc[:, 0:1]
    smi = acc[:, 1:2]
    cnt = acc[:, 2:3]
    media = jnp.where(cnt > 0, summed / cnt, 0.0)
    pv0 = pv_ref[...][:, 0:1]
    velb = vel_ref[...] * pv0 + media
    xn_ref[...] = x_ref[...] + velb
    vo_ref[...] = velb
    hmid = jnp.tanh(jnp.dot(h_ref[...], a_ref[...],
                            preferred_element_type=jnp.float32)
                    + smi * u_ref[...] + b1_ref[...])
    hn_ref[...] = (jnp.dot(hmid, w2_ref[...],
                           preferred_element_type=jnp.float32) + b2_ref[...])


def _tc_final(part, hP, vel8, x8, pv, weights, NP, B=1024):
    grid = (NP // B,)
    row = lambda i: (i, 0)
    fix = lambda i: (0, 0)
    in_specs = [pl.BlockSpec((NC, B, D_AGG), lambda i: (0, i, 0)),
                pl.BlockSpec((B, 128), row), pl.BlockSpec((B, 8), row),
                pl.BlockSpec((B, 8), row), pl.BlockSpec((B, 8), row),
                pl.BlockSpec((128, 128), fix), pl.BlockSpec((1, 128), fix),
                pl.BlockSpec((1, 128), fix), pl.BlockSpec((128, 128), fix),
                pl.BlockSpec((1, 128), fix)]
    out_specs = [pl.BlockSpec((B, 128), row), pl.BlockSpec((B, 8), row),
                 pl.BlockSpec((B, 8), row)]
    return pl.pallas_call(
        _tc_final_body,
        grid=grid,
        in_specs=in_specs,
        out_specs=out_specs,
        out_shape=[jax.ShapeDtypeStruct((NP, 128), jnp.float32),
                   jax.ShapeDtypeStruct((NP, 8), jnp.float32),
                   jax.ShapeDtypeStruct((NP, 8), jnp.float32)],
    )(part, hP, vel8, x8, pv, *weights)


def kernel(h, x, arestas, velocidade, atributos_arestas,
           pxw1, pxb1, pxw2, pxb2,
           pew1, peb1, pew2, peb2,
           phw1, phb1, phw2, phb2,
           pvw1, pvb1, pvw2, pvb2):
    f32 = jnp.float32
    N, ENT = h.shape
    E = arestas.shape[1]
    NP = _round_up(N, NW * 64)
    EP = _round_up(E, NW * 128)

    rows = arestas[0].astype(jnp.int32)
    cols = arestas[1].astype(jnp.int32)

    xt = jnp.pad(x.astype(f32), ((0, 0), (0, 1))).reshape(N * 4)
    rN = jnp.pad(rows[:N], (0, NP - N))
    cN = jnp.pad(cols[:N], (0, NP - N))
    hl, hc, rad1d, sd1d = _sc_gather(h.astype(f32), xt, rN, cN, NP)

    aeP = jnp.pad(atributos_arestas[:N].astype(f32), ((0, NP - N), (0, 0)))
    aeplus = jnp.concatenate(
        [aeP, rad1d[:, None], jnp.zeros((NP, 7), f32)], axis=1)
    IJ = atributos_arestas.shape[1]
    we_aug = jnp.concatenate(
        [pew1[:, 2 * ENT + 1:].T, pew1[:, 2 * ENT].reshape(1, 128),
         jnp.zeros((7, 128), f32)], axis=0)
    hP = jnp.pad(h.astype(f32), ((0, NP - N), (0, 0)))
    mlp_weights = (
        pew1[:, :ENT].T, pew1[:, ENT:2 * ENT].T,
        we_aug,
        peb1.reshape(1, 128),
        pew2.T, peb2.reshape(1, 128),
        pxw1.T, pxb1.reshape(1, 128), pxw2.reshape(1, 128),
        pxb2.reshape(1, 1),
        pvw1.T, pvb1.reshape(1, 128), pvw2.reshape(1, 128),
        pvb2.reshape(1, 1),
    )
    T, pv = _tc_mlp(hl, hc, aeplus, hP, mlp_weights, NP)
    px2d = T[:, 0].reshape(NP // 128, 128)
    sm2d = T[:, 1].reshape(NP // 128, 128)
    sd2d = sd1d.reshape(NP // 128, 128)

    rowsP = jnp.concatenate(
        [rows, jnp.full((EP - E,), NP - 1, jnp.int32)]).reshape(EP // 128, 128)
    colsP = jnp.pad(cols, (0, EP - E))
    zer = jnp.zeros((NP, D_AGG), f32)
    part = _sc_aggregate(px2d, sm2d, sd2d, rowsP, colsP, zer, NP, EP)

    vel8 = jnp.pad(velocidade.astype(f32), ((0, NP - N), (0, 8 - 3)))
    x8 = jnp.pad(x.astype(f32), ((0, NP - N), (0, 8 - 3)))
    fin_weights = (phw1[:, :ENT].T, phw1[:, ENT].reshape(1, 128),
                   phb1.reshape(1, 128), phw2.T, phb2.reshape(1, 128))
    hn, xn8, vo8 = _tc_final(part, hP, vel8, x8, pv, fin_weights, NP)

    return (hn[:N], xn8[:N, :3], vo8[:N, :3])


# speedup vs baseline: 15.3103x; 1.0253x over previous
import functools

import jax
import jax.numpy as jnp
from jax import lax
from jax.experimental import pallas as pl
from jax.experimental.pallas import tpu as pltpu
from jax.experimental.pallas import tpu_sc as plsc

NC = 2
NS = 16
NW = NC * NS

D_TAB = 144
D_AGG = 8
D_TBL = 2


def _round_up(a, b):
    return (a + b - 1) // b * b


def _sc_gather(h, xt, rowsN, colsN, NP):
    E0 = 7 * NP // 10 // 1024 * 1024
    PW = (E0 // NS, (NP - E0) // NS)
    CB = (0, E0)
    pw_max = max(PW)
    n4 = xt.shape[0]
    mesh = plsc.VectorSubcoreMesh(core_axis_name="c", subcore_axis_name="s",
                                  num_cores=NC, num_subcores=NS)

    def _chunks(n):
        out, o = [], 0
        while o < n:
            w = min(128, n - o)
            out.append((o, w))
            o += w
        return out

    @functools.partial(
        pl.kernel, mesh=mesh,
        compiler_params=pltpu.CompilerParams(needs_layout_passes=False),
        out_type=[jax.ShapeDtypeStruct((NP, 128), jnp.float32),
                  jax.ShapeDtypeStruct((NP, 128), jnp.float32),
                  jax.ShapeDtypeStruct((NP,), jnp.float32),
                  jax.ShapeDtypeStruct((NP,), jnp.float32)],
        scratch_types=[pltpu.VMEM((pw_max,), jnp.int32),
                       pltpu.VMEM((pw_max,), jnp.int32),
                       pltpu.VMEM((n4,), jnp.float32),
                       pltpu.VMEM((pw_max // 2, 128), jnp.float32),
                       pltpu.VMEM((pw_max // 2, 128), jnp.float32),
                       pltpu.VMEM((pw_max,), jnp.float32),
                       pltpu.VMEM((pw_max,), jnp.float32),
                       pltpu.VMEM_SHARED((max(E0, NP - E0),), jnp.float32),
                       pltpu.VMEM_SHARED((max(E0, NP - E0),), jnp.float32),
                       pltpu.SemaphoreType.DMA],
    )
    def k(h_hbm, xt_hbm, r_hbm, c_hbm, hl_hbm, hc_hbm, rad_hbm, sd_hbm,
          idxr, idxc, xv, bl, bc, rsv, sdv, rs_sh, sd_sh, sem):
        s = lax.axis_index("s")
        c = lax.axis_index("c")
        pltpu.sync_copy(xt_hbm, xv)

        def work(cbase, per_w):
            half = per_w // 2
            base = cbase + s * per_w
            pltpu.sync_copy(r_hbm.at[pl.ds(base, per_w)],
                            idxr.at[pl.ds(0, per_w)])
            pltpu.sync_copy(c_hbm.at[pl.ds(base, per_w)],
                            idxc.at[pl.ds(0, per_w)])
            cps = []
            for (o, w) in _chunks(half):
                cps.append(pltpu.async_copy(
                    h_hbm.at[idxr.at[pl.ds(o, w)]], bl.at[pl.ds(o, w)], sem))
                cps.append(pltpu.async_copy(
                    h_hbm.at[idxc.at[pl.ds(o, w)]], bc.at[pl.ds(o, w)], sem))

            def body(i, carry):
                r4 = idxr[pl.ds(i * 16, 16)] * 4
                c4 = idxc[pl.ds(i * 16, 16)] * 4
                d0 = plsc.load_gather(xv, [r4]) - plsc.load_gather(xv, [c4])
                d1 = (plsc.load_gather(xv, [r4 + 1])
                      - plsc.load_gather(xv, [c4 + 1]))
                d2 = (plsc.load_gather(xv, [r4 + 2])
                      - plsc.load_gather(xv, [c4 + 2]))
                rsv[pl.ds(i * 16, 16)] = d0 * d0 + d1 * d1 + d2 * d2
                sdv[pl.ds(i * 16, 16)] = d0 + d1 + d2
                return carry

            lax.fori_loop(0, per_w // 16, body, 0)
            for cp in cps:
                cp.wait()
            pltpu.sync_copy(bl.at[pl.ds(0, half)],
                            hl_hbm.at[pl.ds(base, half)])
            pltpu.sync_copy(bc.at[pl.ds(0, half)],
                            hc_hbm.at[pl.ds(base, half)])
            cps = []
            for (o, w) in _chunks(half):
                cps.append(pltpu.async_copy(
                    h_hbm.at[idxr.at[pl.ds(half + o, w)]],
                    bl.at[pl.ds(o, w)], sem))
                cps.append(pltpu.async_copy(
                    h_hbm.at[idxc.at[pl.ds(half + o, w)]],
                    bc.at[pl.ds(o, w)], sem))
            for cp in cps:
                cp.wait()
            pltpu.sync_copy(bl.at[pl.ds(0, half)],
                            hl_hbm.at[pl.ds(base + half, half)])
            pltpu.sync_copy(bc.at[pl.ds(0, half)],
                            hc_hbm.at[pl.ds(base + half, half)])
            pltpu.sync_copy(rsv.at[pl.ds(0, per_w)],
                            rs_sh.at[pl.ds(s * per_w, per_w)])
            pltpu.sync_copy(sdv.at[pl.ds(0, per_w)],
                            sd_sh.at[pl.ds(s * per_w, per_w)])

        @pl.when(c == 0)
        def _():
            work(CB[0], PW[0])

        @pl.when(c == 1)
        def _():
            work(CB[1], PW[1])

        plsc.subcore_barrier()
        for ci in range(NC):
            n_str = PW[ci] * NS // 1024

            @pl.when((c == ci) & (s < n_str))
            def _():
                pltpu.sync_copy(rs_sh.at[pl.ds(s * 1024, 1024)],
                                rad_hbm.at[pl.ds(CB[ci] + s * 1024, 1024)])
                pltpu.sync_copy(sd_sh.at[pl.ds(s * 1024, 1024)],
                                sd_hbm.at[pl.ds(CB[ci] + s * 1024, 1024)])

    return k(h, xt, rowsN, colsN)


def _tc_mlp_body(hl_ref, hc_ref, ae_ref, h_ref,
                 wl_ref, wc_ref, we_ref, b1_ref,
                 w2_ref, b2_ref,
                 xw1_ref, xb1_ref, xw2_ref, xb2_ref,
                 vw1_ref, vb1_ref, vw2_ref, vb2_ref,
                 t_ref, pv_ref):
    a1 = jnp.tanh(
        jnp.dot(hl_ref[...], wl_ref[...], preferred_element_type=jnp.float32)
        + jnp.dot(hc_ref[...], wc_ref[...],
                  preferred_element_type=jnp.float32)
        + jnp.dot(ae_ref[...], we_ref[...], preferred_element_type=jnp.float32)
        + b1_ref[...])
    m = jnp.tanh(jnp.dot(a1, w2_ref[...], preferred_element_type=jnp.float32)
                 + b2_ref[...])
    p1 = jnp.tanh(jnp.dot(m, xw1_ref[...], preferred_element_type=jnp.float32)
                  + xb1_ref[...])
    px = jnp.tanh(jnp.sum(p1 * xw2_ref[...], axis=1, keepdims=True)
                  + xb2_ref[...])
    s_m = jnp.sum(m, axis=1, keepdims=True)
    v1 = jnp.tanh(jnp.dot(h_ref[...], vw1_ref[...],
                          preferred_element_type=jnp.float32) + vb1_ref[...])
    pv = jnp.sum(v1 * vw2_ref[...], axis=1, keepdims=True) + vb2_ref[...]
    col = lax.broadcasted_iota(jnp.int32, t_ref.shape, 1)
    t_ref[...] = jnp.where(col == 0, px,
                           jnp.where(col == 1, s_m, 0.0))
    pv_ref[...] = jnp.broadcast_to(pv, pv_ref.shape)


def _tc_mlp(hl, hc, aeplus, hP, weights, NP, B=2048):
    grid = (NP // B,)
    row = lambda i: (i, 0)
    fix = lambda i: (0, 0)

    def spec(shape, imap):
        return pl.BlockSpec(shape, imap)

    in_specs = [spec((B, 128), row), spec((B, 128), row),
                spec((B, 24), row), spec((B, 128), row)]
    for wshape in [(128, 128), (128, 128), (24, 128), (1, 128),
                   (128, 128), (1, 128),
                   (128, 128), (1, 128), (1, 128), (1, 1),
                   (128, 128), (1, 128), (1, 128), (1, 1)]:
        in_specs.append(spec(wshape, fix))
    out_specs = [spec((B, 8), row), spec((B, 8), row)]
    return pl.pallas_call(
        _tc_mlp_body,
        grid=grid,
        in_specs=in_specs,
        out_specs=out_specs,
        out_shape=[jax.ShapeDtypeStruct((NP, 8), jnp.float32),
                   jax.ShapeDtypeStruct((NP, 8), jnp.float32)],
    )(hl, hc, aeplus, hP, *weights)


def _sc_aggregate(px2d, sm2d, sd2d, rows2d, cols1d, zeros_hbm, NP, EP):
    ep_w = EP // NW
    KR = ep_w // 128
    stripe = NP // NS
    mesh = plsc.VectorSubcoreMesh(core_axis_name="c", subcore_axis_name="s",
                                  num_cores=NC, num_subcores=NS)

    @functools.partial(
        pl.kernel, mesh=mesh,
        compiler_params=pltpu.CompilerParams(use_tc_tiling_on_sc=False,
                                             needs_layout_passes=False),
        out_type=jax.ShapeDtypeStruct((NC, NP, D_AGG), jnp.float32),
        scratch_types=[pltpu.VMEM((KR, 128), jnp.int32),
                       pltpu.VMEM((ep_w,), jnp.int32),
                       pltpu.VMEM((NP // 128, 128), jnp.float32),
                       pltpu.VMEM((NP // 128, 128), jnp.float32),
                       pltpu.VMEM((NP // 128, 128), jnp.float32),
                       pltpu.VMEM((ep_w, D_AGG), jnp.float32),
                       pltpu.VMEM_SHARED((NP, D_AGG), jnp.float32),
                       pltpu.SemaphoreType.DMA],
    )
    def k(px_hbm, sm_hbm, sd_hbm, r_hbm, c_hbm, z_hbm, out_hbm, idxr, idxc,
          tvs, tvm, tvd, gbuf, acc, sem):
        s = lax.axis_index("s")
        c = lax.axis_index("c")
        wid = s * NC + c
        pltpu.sync_copy(z_hbm.at[pl.ds(s * stripe, stripe)],
                        acc.at[pl.ds(s * stripe, stripe)])
        pltpu.sync_copy(px_hbm, tvs)
        pltpu.sync_copy(sm_hbm, tvm)
        pltpu.sync_copy(sd_hbm, tvd)
        pltpu.sync_copy(c_hbm.at[pl.ds(wid * ep_w, ep_w)], idxc)
        pltpu.sync_copy(r_hbm.at[pl.ds(wid * KR, KR)], idxr)
        lane = lax.iota(jnp.int32, 16)
        zero16 = jnp.zeros((16,), jnp.int32)
        one16 = jnp.full((16,), 1, jnp.int32)
        two16 = jnp.full((16,), 2, jnp.int32)
        ones_f = jnp.full((16,), 1.0, jnp.float32)

        def body(i, carry):
            ci = idxc[pl.ds(i * 16, 16)]
            tr = lax.shift_right_logical(ci, 7)
            tc_ = jnp.bitwise_and(ci, 127)
            sv = (plsc.load_gather(tvs, [tr, tc_])
                  * plsc.load_gather(tvd, [tr, tc_]))
            sm = plsc.load_gather(tvm, [tr, tc_])
            e16 = lane + i * 16
            plsc.store_scatter(gbuf, [e16, zero16], sv)
            plsc.store_scatter(gbuf, [e16, one16], sm)
            plsc.store_scatter(gbuf, [e16, two16], ones_f)
            return carry

        lax.fori_loop(0, ep_w // 16, body, 0)
        plsc.subcore_barrier()
        cps = [pltpu.async_copy(gbuf.at[pl.ds(j * 128, 128)],
                                acc.at[idxr.at[j]], sem, add=True)
               for j in range(KR)]
        for cp in cps:
            cp.wait()
        plsc.subcore_barrier()
        pltpu.sync_copy(acc.at[pl.ds(s * stripe, stripe)],
                        out_hbm.at[c].at[pl.ds(s * stripe, stripe)])

    return k(px2d, sm2d, sd2d, rows2d, cols1d, zeros_hbm)


def _tc_final_body(part_ref, h_ref, vel_ref, x_ref, pv_ref,
                   a_ref, u_ref, b1_ref, w2_ref, b2_ref,
                   hn_ref, xn_ref, vo_ref):
    acc = part_ref[0] + part_ref[1]
    summed = acc[:, 0:1]
    smi = acc[:, 1:2]
    cnt = acc[:, 2:3]
    media = jnp.where(cnt > 0, summed / cnt, 0.0)
    pv0 = pv_ref[...][:, 0:1]
    velb = vel_ref[...] * pv0 + media
    xn_ref[...] = x_ref[...] + velb
    vo_ref[...] = velb
    hmid = jnp.tanh(jnp.dot(h_ref[...], a_ref[...],
                            preferred_element_type=jnp.float32)
                    + smi * u_ref[...] + b1_ref[...])
    hn_ref[...] = (jnp.dot(hmid, w2_ref[...],
                           preferred_element_type=jnp.float32) + b2_ref[...])


def _tc_final(part, hP, vel8, x8, pv, weights, NP, N_OUT, B=1024):
    grid = (NP // B,)
    row = lambda i: (i, 0)
    fix = lambda i: (0, 0)
    in_specs = [pl.BlockSpec((NC, B, D_AGG), lambda i: (0, i, 0)),
                pl.BlockSpec((B, 128), row), pl.BlockSpec((B, 8), row),
                pl.BlockSpec((B, 8), row), pl.BlockSpec((B, 8), row),
                pl.BlockSpec((128, 128), fix), pl.BlockSpec((1, 128), fix),
                pl.BlockSpec((1, 128), fix), pl.BlockSpec((128, 128), fix),
                pl.BlockSpec((1, 128), fix)]
    out_specs = [pl.BlockSpec((B, 128), row), pl.BlockSpec((B, 8), row),
                 pl.BlockSpec((B, 8), row)]
    return pl.pallas_call(
        _tc_final_body,
        grid=grid,
        in_specs=in_specs,
        out_specs=out_specs,
        out_shape=[jax.ShapeDtypeStruct((N_OUT, 128), jnp.float32),
                   jax.ShapeDtypeStruct((N_OUT, 8), jnp.float32),
                   jax.ShapeDtypeStruct((N_OUT, 8), jnp.float32)],
    )(part, hP, vel8, x8, pv, *weights)


def kernel(h, x, arestas, velocidade, atributos_arestas,
           pxw1, pxb1, pxw2, pxb2,
           pew1, peb1, pew2, peb2,
           phw1, phb1, phw2, phb2,
           pvw1, pvb1, pvw2, pvb2):
    f32 = jnp.float32
    N, ENT = h.shape
    E = arestas.shape[1]
    NP = _round_up(N, NW * 64)
    EP = _round_up(E, NW * 128)

    rows = arestas[0].astype(jnp.int32)
    cols = arestas[1].astype(jnp.int32)

    xt = jnp.pad(x.astype(f32), ((0, 0), (0, 1))).reshape(N * 4)
    rN = jnp.pad(rows[:N], (0, NP - N))
    cN = jnp.pad(cols[:N], (0, NP - N))
    hl, hc, rad1d, sd1d = _sc_gather(h.astype(f32), xt, rN, cN, NP)

    aeP = jnp.pad(atributos_arestas[:N].astype(f32), ((0, NP - N), (0, 0)))
    aeplus = jnp.concatenate(
        [aeP, rad1d[:, None], jnp.zeros((NP, 7), f32)], axis=1)
    IJ = atributos_arestas.shape[1]
    we_aug = jnp.concatenate(
        [pew1[:, 2 * ENT + 1:].T, pew1[:, 2 * ENT].reshape(1, 128),
         jnp.zeros((7, 128), f32)], axis=0)
    hP = jnp.pad(h.astype(f32), ((0, NP - N), (0, 0)))
    mlp_weights = (
        pew1[:, :ENT].T, pew1[:, ENT:2 * ENT].T,
        we_aug,
        peb1.reshape(1, 128),
        pew2.T, peb2.reshape(1, 128),
        pxw1.T, pxb1.reshape(1, 128), pxw2.reshape(1, 128),
        pxb2.reshape(1, 1),
        pvw1.T, pvb1.reshape(1, 128), pvw2.reshape(1, 128),
        pvb2.reshape(1, 1),
    )
    T, pv = _tc_mlp(hl, hc, aeplus, hP, mlp_weights, NP)
    px2d = T[:, 0].reshape(NP // 128, 128)
    sm2d = T[:, 1].reshape(NP // 128, 128)
    sd2d = sd1d.reshape(NP // 128, 128)

    rowsP = jnp.concatenate(
        [rows, jnp.full((EP - E,), NP - 1, jnp.int32)]).reshape(EP // 128, 128)
    colsP = jnp.pad(cols, (0, EP - E))
    zer = jnp.zeros((NP, D_AGG), f32)
    part = _sc_aggregate(px2d, sm2d, sd2d, rowsP, colsP, zer, NP, EP)

    vel8 = jnp.pad(velocidade.astype(f32), ((0, NP - N), (0, 8 - 3)))
    x8 = jnp.pad(x.astype(f32), ((0, NP - N), (0, 8 - 3)))
    fin_weights = (phw1[:, :ENT].T, phw1[:, ENT].reshape(1, 128),
                   phb1.reshape(1, 128), phw2.T, phb2.reshape(1, 128))
    hn, xn8, vo8 = _tc_final(part, hP, vel8, x8, pv, fin_weights, NP, N)

    return (hn, xn8[:, :3], vo8[:, :3])


# speedup vs baseline: 15.8254x; 1.0336x over previous
import functools

import jax
import jax.numpy as jnp
from jax import lax
from jax.experimental import pallas as pl
from jax.experimental.pallas import tpu as pltpu
from jax.experimental.pallas import tpu_sc as plsc

NC = 2
NS = 16
NW = NC * NS

D_TAB = 144
D_AGG = 8
D_TBL = 2


def _round_up(a, b):
    return (a + b - 1) // b * b


def _sc_gather(h, xt, rowsN, colsN, NP):
    E0 = 5 * NP // 10 // 1024 * 1024
    PW = (E0 // NS, (NP - E0) // NS)
    CB = (0, E0)
    pw_max = max(PW)
    n4 = xt.shape[0]
    mesh = plsc.VectorSubcoreMesh(core_axis_name="c", subcore_axis_name="s",
                                  num_cores=NC, num_subcores=NS)

    def _chunks(n):
        out, o = [], 0
        while o < n:
            w = min(128, n - o)
            out.append((o, w))
            o += w
        return out

    @functools.partial(
        pl.kernel, mesh=mesh,
        compiler_params=pltpu.CompilerParams(needs_layout_passes=False),
        out_type=[jax.ShapeDtypeStruct((NP, 128), jnp.float32),
                  jax.ShapeDtypeStruct((NP, 128), jnp.float32),
                  jax.ShapeDtypeStruct((NP,), jnp.float32),
                  jax.ShapeDtypeStruct((NP,), jnp.float32)],
        scratch_types=[pltpu.VMEM((pw_max,), jnp.int32),
                       pltpu.VMEM((pw_max,), jnp.int32),
                       pltpu.VMEM((n4,), jnp.float32),
                       pltpu.VMEM((pw_max // 2, 128), jnp.float32),
                       pltpu.VMEM((pw_max // 2, 128), jnp.float32),
                       pltpu.VMEM((pw_max,), jnp.float32),
                       pltpu.VMEM((pw_max,), jnp.float32),
                       pltpu.VMEM_SHARED((max(E0, NP - E0),), jnp.float32),
                       pltpu.VMEM_SHARED((max(E0, NP - E0),), jnp.float32),
                       pltpu.VMEM_SHARED((n4,), jnp.float32),
                       pltpu.SemaphoreType.DMA],
    )
    def k(h_hbm, xt_hbm, r_hbm, c_hbm, hl_hbm, hc_hbm, rad_hbm, sd_hbm,
          idxr, idxc, xv, bl, bc, rsv, sdv, rs_sh, sd_sh, xs, sem):
        s = lax.axis_index("s")
        c = lax.axis_index("c")

        @pl.when(s == 0)
        def _():
            pltpu.sync_copy(xt_hbm, xs)

        plsc.subcore_barrier()
        pltpu.sync_copy(xs, xv)

        def work(cbase, per_w):
            half = per_w // 2
            base = cbase + s * per_w
            pltpu.sync_copy(r_hbm.at[pl.ds(base, per_w)],
                            idxr.at[pl.ds(0, per_w)])
            pltpu.sync_copy(c_hbm.at[pl.ds(base, per_w)],
                            idxc.at[pl.ds(0, per_w)])
            cps = []
            for (o, w) in _chunks(half):
                cps.append(pltpu.async_copy(
                    h_hbm.at[idxr.at[pl.ds(o, w)]], bl.at[pl.ds(o, w)], sem))
                cps.append(pltpu.async_copy(
                    h_hbm.at[idxc.at[pl.ds(o, w)]], bc.at[pl.ds(o, w)], sem))

            def body(i, carry):
                r4 = idxr[pl.ds(i * 16, 16)] * 4
                c4 = idxc[pl.ds(i * 16, 16)] * 4
                d0 = plsc.load_gather(xv, [r4]) - plsc.load_gather(xv, [c4])
                d1 = (plsc.load_gather(xv, [r4 + 1])
                      - plsc.load_gather(xv, [c4 + 1]))
                d2 = (plsc.load_gather(xv, [r4 + 2])
                      - plsc.load_gather(xv, [c4 + 2]))
                rsv[pl.ds(i * 16, 16)] = d0 * d0 + d1 * d1 + d2 * d2
                sdv[pl.ds(i * 16, 16)] = d0 + d1 + d2
                return carry

            lax.fori_loop(0, per_w // 16, body, 0)
            for cp in cps:
                cp.wait()
            pltpu.sync_copy(bl.at[pl.ds(0, half)],
                            hl_hbm.at[pl.ds(base, half)])
            pltpu.sync_copy(bc.at[pl.ds(0, half)],
                            hc_hbm.at[pl.ds(base, half)])
            cps = []
            for (o, w) in _chunks(half):
                cps.append(pltpu.async_copy(
                    h_hbm.at[idxr.at[pl.ds(half + o, w)]],
                    bl.at[pl.ds(o, w)], sem))
                cps.append(pltpu.async_copy(
                    h_hbm.at[idxc.at[pl.ds(half + o, w)]],
                    bc.at[pl.ds(o, w)], sem))
            for cp in cps:
                cp.wait()
            pltpu.sync_copy(bl.at[pl.ds(0, half)],
                            hl_hbm.at[pl.ds(base + half, half)])
            pltpu.sync_copy(bc.at[pl.ds(0, half)],
                            hc_hbm.at[pl.ds(base + half, half)])
            pltpu.sync_copy(rsv.at[pl.ds(0, per_w)],
                            rs_sh.at[pl.ds(s * per_w, per_w)])
            pltpu.sync_copy(sdv.at[pl.ds(0, per_w)],
                            sd_sh.at[pl.ds(s * per_w, per_w)])

        @pl.when(c == 0)
        def _():
            work(CB[0], PW[0])

        @pl.when(c == 1)
        def _():
            work(CB[1], PW[1])

        plsc.subcore_barrier()
        for ci in range(NC):
            n_str = PW[ci] * NS // 1024

            @pl.when((c == ci) & (s < n_str))
            def _():
                pltpu.sync_copy(rs_sh.at[pl.ds(s * 1024, 1024)],
                                rad_hbm.at[pl.ds(CB[ci] + s * 1024, 1024)])
                pltpu.sync_copy(sd_sh.at[pl.ds(s * 1024, 1024)],
                                sd_hbm.at[pl.ds(CB[ci] + s * 1024, 1024)])

    return k(h, xt, rowsN, colsN)


def _tc_mlp_body(hl_ref, hc_ref, ae_ref, h_ref,
                 wl_ref, wc_ref, we_ref, b1_ref,
                 w2_ref, b2_ref,
                 xw1_ref, xb1_ref, xw2_ref, xb2_ref,
                 vw1_ref, vb1_ref, vw2_ref, vb2_ref,
                 t_ref, pv_ref):
    a1 = jnp.tanh(
        jnp.dot(hl_ref[...], wl_ref[...], preferred_element_type=jnp.float32)
        + jnp.dot(hc_ref[...], wc_ref[...],
                  preferred_element_type=jnp.float32)
        + jnp.dot(ae_ref[...], we_ref[...], preferred_element_type=jnp.float32)
        + b1_ref[...])
    m = jnp.tanh(jnp.dot(a1, w2_ref[...], preferred_element_type=jnp.float32)
                 + b2_ref[...])
    p1 = jnp.tanh(jnp.dot(m, xw1_ref[...], preferred_element_type=jnp.float32)
                  + xb1_ref[...])
    px = jnp.tanh(jnp.sum(p1 * xw2_ref[...], axis=1, keepdims=True)
                  + xb2_ref[...])
    s_m = jnp.sum(m, axis=1, keepdims=True)
    v1 = jnp.tanh(jnp.dot(h_ref[...], vw1_ref[...],
                          preferred_element_type=jnp.float32) + vb1_ref[...])
    pv = jnp.sum(v1 * vw2_ref[...], axis=1, keepdims=True) + vb2_ref[...]
    col = lax.broadcasted_iota(jnp.int32, t_ref.shape, 1)
    t_ref[...] = jnp.where(col == 0, px,
                           jnp.where(col == 1, s_m, 0.0))
    pv_ref[...] = jnp.broadcast_to(pv, pv_ref.shape)


def _tc_mlp(hl, hc, aeplus, hP, weights, NP, B=1024):
    grid = (NP // B,)
    row = lambda i: (i, 0)
    fix = lambda i: (0, 0)

    def spec(shape, imap):
        return pl.BlockSpec(shape, imap)

    in_specs = [spec((B, 128), row), spec((B, 128), row),
                spec((B, 24), row), spec((B, 128), row)]
    for wshape in [(128, 128), (128, 128), (24, 128), (1, 128),
                   (128, 128), (1, 128),
                   (128, 128), (1, 128), (1, 128), (1, 1),
                   (128, 128), (1, 128), (1, 128), (1, 1)]:
        in_specs.append(spec(wshape, fix))
    out_specs = [spec((B, 8), row), spec((B, 8), row)]
    return pl.pallas_call(
        _tc_mlp_body,
        grid=grid,
        in_specs=in_specs,
        out_specs=out_specs,
        out_shape=[jax.ShapeDtypeStruct((NP, 8), jnp.float32),
                   jax.ShapeDtypeStruct((NP, 8), jnp.float32)],
    )(hl, hc, aeplus, hP, *weights)


def _sc_aggregate(px2d, sm2d, sd2d, rows2d, cols1d, zeros_hbm, NP, EP):
    ep_w = EP // NW
    KR = ep_w // 128
    stripe = NP // NS
    mesh = plsc.VectorSubcoreMesh(core_axis_name="c", subcore_axis_name="s",
                                  num_cores=NC, num_subcores=NS)

    @functools.partial(
        pl.kernel, mesh=mesh,
        compiler_params=pltpu.CompilerParams(use_tc_tiling_on_sc=False,
                                             needs_layout_passes=False),
        out_type=jax.ShapeDtypeStruct((NC, NP, D_AGG), jnp.float32),
        scratch_types=[pltpu.VMEM((KR, 128), jnp.int32),
                       pltpu.VMEM((ep_w,), jnp.int32),
                       pltpu.VMEM((NP // 128, 128), jnp.float32),
                       pltpu.VMEM((NP // 128, 128), jnp.float32),
                       pltpu.VMEM((NP // 128, 128), jnp.float32),
                       pltpu.VMEM((ep_w, D_AGG), jnp.float32),
                       pltpu.VMEM_SHARED((NP, D_AGG), jnp.float32),
                       pltpu.SemaphoreType.DMA],
    )
    def k(px_hbm, sm_hbm, sd_hbm, r_hbm, c_hbm, z_hbm, out_hbm, idxr, idxc,
          tvs, tvm, tvd, gbuf, acc, sem):
        s = lax.axis_index("s")
        c = lax.axis_index("c")
        wid = s * NC + c
        pltpu.sync_copy(z_hbm.at[pl.ds(s * stripe, stripe)],
                        acc.at[pl.ds(s * stripe, stripe)])
        pltpu.sync_copy(px_hbm, tvs)
        pltpu.sync_copy(sm_hbm, tvm)
        pltpu.sync_copy(sd_hbm, tvd)
        pltpu.sync_copy(c_hbm.at[pl.ds(wid * ep_w, ep_w)], idxc)
        pltpu.sync_copy(r_hbm.at[pl.ds(wid * KR, KR)], idxr)
        lane = lax.iota(jnp.int32, 16)
        zero16 = jnp.zeros((16,), jnp.int32)
        one16 = jnp.full((16,), 1, jnp.int32)
        two16 = jnp.full((16,), 2, jnp.int32)
        ones_f = jnp.full((16,), 1.0, jnp.float32)

        def body(i, carry):
            ci = idxc[pl.ds(i * 16, 16)]
            tr = lax.shift_right_logical(ci, 7)
            tc_ = jnp.bitwise_and(ci, 127)
            sv = (plsc.load_gather(tvs, [tr, tc_])
                  * plsc.load_gather(tvd, [tr, tc_]))
            sm = plsc.load_gather(tvm, [tr, tc_])
            e16 = lane + i * 16
            plsc.store_scatter(gbuf, [e16, zero16], sv)
            plsc.store_scatter(gbuf, [e16, one16], sm)
            plsc.store_scatter(gbuf, [e16, two16], ones_f)
            return carry

        lax.fori_loop(0, ep_w // 16, body, 0)
        plsc.subcore_barrier()
        cps = [pltpu.async_copy(gbuf.at[pl.ds(j * 128, 128)],
                                acc.at[idxr.at[j]], sem, add=True)
               for j in range(KR)]
        for cp in cps:
            cp.wait()
        plsc.subcore_barrier()
        pltpu.sync_copy(acc.at[pl.ds(s * stripe, stripe)],
                        out_hbm.at[c].at[pl.ds(s * stripe, stripe)])

    return k(px2d, sm2d, sd2d, rows2d, cols1d, zeros_hbm)


def _tc_final_body(part_ref, h_ref, vel_ref, x_ref, pv_ref,
                   a_ref, u_ref, b1_ref, w2_ref, b2_ref,
                   hn_ref, xn_ref, vo_ref):
    acc = part_ref[0] + part_ref[1]
    summed = acc[:, 0:1]
    smi = acc[:, 1:2]
    cnt = acc[:, 2:3]
    media = jnp.where(cnt > 0, summed / cnt, 0.0)
    pv0 = pv_ref[...][:, 0:1]
    velb = vel_ref[...] * pv0 + media
    xn_ref[...] = x_ref[...] + velb
    vo_ref[...] = velb
    hmid = jnp.tanh(jnp.dot(h_ref[...], a_ref[...],
                            preferred_element_type=jnp.float32)
                    + smi * u_ref[...] + b1_ref[...])
    hn_ref[...] = (jnp.dot(hmid, w2_ref[...],
                           preferred_element_type=jnp.float32) + b2_ref[...])


def _tc_final(part, hP, vel8, x8, pv, weights, NP, N_OUT, B=1024):
    grid = (NP // B,)
    row = lambda i: (i, 0)
    fix = lambda i: (0, 0)
    in_specs = [pl.BlockSpec((NC, B, D_AGG), lambda i: (0, i, 0)),
                pl.BlockSpec((B, 128), row), pl.BlockSpec((B, 8), row),
                pl.BlockSpec((B, 8), row), pl.BlockSpec((B, 8), row),
                pl.BlockSpec((128, 128), fix), pl.BlockSpec((1, 128), fix),
                pl.BlockSpec((1, 128), fix), pl.BlockSpec((128, 128), fix),
                pl.BlockSpec((1, 128), fix)]
    out_specs = [pl.BlockSpec((B, 128), row), pl.BlockSpec((B, 8), row),
                 pl.BlockSpec((B, 8), row)]
    return pl.pallas_call(
        _tc_final_body,
        grid=grid,
        in_specs=in_specs,
        out_specs=out_specs,
        out_shape=[jax.ShapeDtypeStruct((N_OUT, 128), jnp.float32),
                   jax.ShapeDtypeStruct((N_OUT, 8), jnp.float32),
                   jax.ShapeDtypeStruct((N_OUT, 8), jnp.float32)],
    )(part, hP, vel8, x8, pv, *weights)


def kernel(h, x, arestas, velocidade, atributos_arestas,
           pxw1, pxb1, pxw2, pxb2,
           pew1, peb1, pew2, peb2,
           phw1, phb1, phw2, phb2,
           pvw1, pvb1, pvw2, pvb2):
    f32 = jnp.float32
    N, ENT = h.shape
    E = arestas.shape[1]
    NP = _round_up(N, NW * 64)
    EP = _round_up(E, NW * 128)

    rows = arestas[0].astype(jnp.int32)
    cols = arestas[1].astype(jnp.int32)

    xt = jnp.pad(x.astype(f32), ((0, 0), (0, 1))).reshape(N * 4)
    rN = jnp.pad(rows[:N], (0, NP - N))
    cN = jnp.pad(cols[:N], (0, NP - N))
    hl, hc, rad1d, sd1d = _sc_gather(h.astype(f32), xt, rN, cN, NP)

    aeP = jnp.pad(atributos_arestas[:N].astype(f32), ((0, NP - N), (0, 0)))
    aeplus = jnp.concatenate(
        [aeP, rad1d[:, None], jnp.zeros((NP, 7), f32)], axis=1)
    IJ = atributos_arestas.shape[1]
    we_aug = jnp.concatenate(
        [pew1[:, 2 * ENT + 1:].T, pew1[:, 2 * ENT].reshape(1, 128),
         jnp.zeros((7, 128), f32)], axis=0)
    hP = jnp.pad(h.astype(f32), ((0, NP - N), (0, 0)))
    mlp_weights = (
        pew1[:, :ENT].T, pew1[:, ENT:2 * ENT].T,
        we_aug,
        peb1.reshape(1, 128),
        pew2.T, peb2.reshape(1, 128),
        pxw1.T, pxb1.reshape(1, 128), pxw2.reshape(1, 128),
        pxb2.reshape(1, 1),
        pvw1.T, pvb1.reshape(1, 128), pvw2.reshape(1, 128),
        pvb2.reshape(1, 1),
    )
    T, pv = _tc_mlp(hl, hc, aeplus, hP, mlp_weights, NP)
    px2d = T[:, 0].reshape(NP // 128, 128)
    sm2d = T[:, 1].reshape(NP // 128, 128)
    sd2d = sd1d.reshape(NP // 128, 128)

    rowsP = jnp.concatenate(
        [rows, jnp.full((EP - E,), NP - 1, jnp.int32)]).reshape(EP // 128, 128)
    colsP = jnp.pad(cols, (0, EP - E))
    zer = jnp.zeros((NP, D_AGG), f32)
    part = _sc_aggregate(px2d, sm2d, sd2d, rowsP, colsP, zer, NP, EP)

    vel8 = jnp.pad(velocidade.astype(f32), ((0, NP - N), (0, 8 - 3)))
    x8 = jnp.pad(x.astype(f32), ((0, NP - N), (0, 8 - 3)))
    fin_weights = (phw1[:, :ENT].T, phw1[:, ENT].reshape(1, 128),
                   phb1.reshape(1, 128), phw2.T, phb2.reshape(1, 128))
    hn, xn8, vo8 = _tc_final(part, hP, vel8, x8, pv, fin_weights, NP, N)

    return (hn, xn8[:, :3], vo8[:, :3])


# speedup vs baseline: 15.8403x; 1.0009x over previous
import functools

import jax
import jax.numpy as jnp
from jax import lax
from jax.experimental import pallas as pl
from jax.experimental.pallas import tpu as pltpu
from jax.experimental.pallas import tpu_sc as plsc

NC = 2
NS = 16
NW = NC * NS

D_AGG = 8


def _round_up(a, b):
    return (a + b - 1) // b * b


def _sc_gather(h, xt, rowsN, colsN, NP):
    E0 = 5 * NP // 10 // 1024 * 1024
    PW = (E0 // NS, (NP - E0) // NS)
    CB = (0, E0)
    pw_max = max(PW)
    n4 = xt.shape[0]
    mesh = plsc.VectorSubcoreMesh(core_axis_name="c", subcore_axis_name="s",
                                  num_cores=NC, num_subcores=NS)

    def _chunks(n):
        out, o = [], 0
        while o < n:
            w = min(128, n - o)
            out.append((o, w))
            o += w
        return out

    @functools.partial(
        pl.kernel, mesh=mesh,
        compiler_params=pltpu.CompilerParams(needs_layout_passes=False),
        out_type=[jax.ShapeDtypeStruct((NP, 128), jnp.float32),
                  jax.ShapeDtypeStruct((NP, 128), jnp.float32),
                  jax.ShapeDtypeStruct((NP,), jnp.float32),
                  jax.ShapeDtypeStruct((NP,), jnp.float32)],
        scratch_types=[pltpu.VMEM((pw_max,), jnp.int32),
                       pltpu.VMEM((pw_max,), jnp.int32),
                       pltpu.VMEM((n4,), jnp.float32),
                       pltpu.VMEM((pw_max // 2, 128), jnp.float32),
                       pltpu.VMEM((pw_max // 2, 128), jnp.float32),
                       pltpu.VMEM((pw_max,), jnp.float32),
                       pltpu.VMEM((pw_max,), jnp.float32),
                       pltpu.VMEM_SHARED((max(E0, NP - E0),), jnp.float32),
                       pltpu.VMEM_SHARED((max(E0, NP - E0),), jnp.float32),
                       pltpu.VMEM_SHARED((n4,), jnp.float32),
                       pltpu.SemaphoreType.DMA],
    )
    def k(h_hbm, xt_hbm, r_hbm, c_hbm, hl_hbm, hc_hbm, rad_hbm, sd_hbm,
          idxr, idxc, xv, bl, bc, rsv, sdv, rs_sh, sd_sh, xs, sem):
        s = lax.axis_index("s")
        c = lax.axis_index("c")

        @pl.when(s == 0)
        def _():
            pltpu.sync_copy(xt_hbm, xs)

        plsc.subcore_barrier()
        pltpu.sync_copy(xs, xv)

        def work(cbase, per_w):
            half = per_w // 2
            base = cbase + s * per_w
            pltpu.sync_copy(r_hbm.at[pl.ds(base, per_w)],
                            idxr.at[pl.ds(0, per_w)])
            pltpu.sync_copy(c_hbm.at[pl.ds(base, per_w)],
                            idxc.at[pl.ds(0, per_w)])
            cps = []
            for (o, w) in _chunks(half):
                cps.append(pltpu.async_copy(
                    h_hbm.at[idxr.at[pl.ds(o, w)]], bl.at[pl.ds(o, w)], sem))
                cps.append(pltpu.async_copy(
                    h_hbm.at[idxc.at[pl.ds(o, w)]], bc.at[pl.ds(o, w)], sem))

            def body(i, carry):
                r4 = idxr[pl.ds(i * 16, 16)] * 4
                c4 = idxc[pl.ds(i * 16, 16)] * 4
                d0 = plsc.load_gather(xv, [r4]) - plsc.load_gather(xv, [c4])
                d1 = (plsc.load_gather(xv, [r4 + 1])
                      - plsc.load_gather(xv, [c4 + 1]))
                d2 = (plsc.load_gather(xv, [r4 + 2])
                      - plsc.load_gather(xv, [c4 + 2]))
                rsv[pl.ds(i * 16, 16)] = d0 * d0 + d1 * d1 + d2 * d2
                sdv[pl.ds(i * 16, 16)] = d0 + d1 + d2
                return carry

            lax.fori_loop(0, per_w // 16, body, 0)
            for cp in cps:
                cp.wait()
            pltpu.sync_copy(bl.at[pl.ds(0, half)],
                            hl_hbm.at[pl.ds(base, half)])
            pltpu.sync_copy(bc.at[pl.ds(0, half)],
                            hc_hbm.at[pl.ds(base, half)])
            cps = []
            for (o, w) in _chunks(half):
                cps.append(pltpu.async_copy(
                    h_hbm.at[idxr.at[pl.ds(half + o, w)]],
                    bl.at[pl.ds(o, w)], sem))
                cps.append(pltpu.async_copy(
                    h_hbm.at[idxc.at[pl.ds(half + o, w)]],
                    bc.at[pl.ds(o, w)], sem))
            for cp in cps:
                cp.wait()
            pltpu.sync_copy(bl.at[pl.ds(0, half)],
                            hl_hbm.at[pl.ds(base + half, half)])
            pltpu.sync_copy(bc.at[pl.ds(0, half)],
                            hc_hbm.at[pl.ds(base + half, half)])
            pltpu.sync_copy(rsv.at[pl.ds(0, per_w)],
                            rs_sh.at[pl.ds(s * per_w, per_w)])
            pltpu.sync_copy(sdv.at[pl.ds(0, per_w)],
                            sd_sh.at[pl.ds(s * per_w, per_w)])

        @pl.when(c == 0)
        def _():
            work(CB[0], PW[0])

        @pl.when(c == 1)
        def _():
            work(CB[1], PW[1])

        plsc.subcore_barrier()
        for ci in range(NC):
            n_str = PW[ci] * NS // 1024

            @pl.when((c == ci) & (s < n_str))
            def _():
                pltpu.sync_copy(rs_sh.at[pl.ds(s * 1024, 1024)],
                                rad_hbm.at[pl.ds(CB[ci] + s * 1024, 1024)])
                pltpu.sync_copy(sd_sh.at[pl.ds(s * 1024, 1024)],
                                sd_hbm.at[pl.ds(CB[ci] + s * 1024, 1024)])

    return k(h, xt, rowsN, colsN)


def _tc_mlp_body(hl_ref, hc_ref, ae_ref, h_ref,
                 wl_ref, wc_ref, we_ref, b1_ref,
                 w2_ref, b2_ref,
                 xw1_ref, xb1_ref, xw2_ref, xb2_ref,
                 vw1_ref, vb1_ref, vw2_ref, vb2_ref,
                 t_ref, pv_ref):
    a1 = jnp.tanh(
        jnp.dot(hl_ref[...], wl_ref[...], preferred_element_type=jnp.float32)
        + jnp.dot(hc_ref[...], wc_ref[...],
                  preferred_element_type=jnp.float32)
        + jnp.dot(ae_ref[...], we_ref[...], preferred_element_type=jnp.float32)
        + b1_ref[...])
    m = jnp.tanh(jnp.dot(a1, w2_ref[...], preferred_element_type=jnp.float32)
                 + b2_ref[...])
    p1 = jnp.tanh(jnp.dot(m, xw1_ref[...], preferred_element_type=jnp.float32)
                  + xb1_ref[...])
    px = jnp.tanh(jnp.sum(p1 * xw2_ref[...], axis=1, keepdims=True)
                  + xb2_ref[...])
    s_m = jnp.sum(m, axis=1, keepdims=True)
    v1 = jnp.tanh(jnp.dot(h_ref[...], vw1_ref[...],
                          preferred_element_type=jnp.float32) + vb1_ref[...])
    pv = jnp.sum(v1 * vw2_ref[...], axis=1, keepdims=True) + vb2_ref[...]
    col = lax.broadcasted_iota(jnp.int32, t_ref.shape, 1)
    t_ref[...] = jnp.where(col == 0, px,
                           jnp.where(col == 1, s_m, 0.0))
    pv_ref[...] = jnp.broadcast_to(pv, pv_ref.shape)


def _tc_mlp(hl, hc, aeplus, hP, weights, NP, B=1024):
    grid = (NP // B,)
    row = lambda i: (i, 0)
    fix = lambda i: (0, 0)

    def spec(shape, imap):
        return pl.BlockSpec(shape, imap)

    in_specs = [spec((B, 128), row), spec((B, 128), row),
                spec((B, 24), row), spec((B, 128), row)]
    for wshape in [(128, 128), (128, 128), (24, 128), (1, 128),
                   (128, 128), (1, 128),
                   (128, 128), (1, 128), (1, 128), (1, 1),
                   (128, 128), (1, 128), (1, 128), (1, 1)]:
        in_specs.append(spec(wshape, fix))
    out_specs = [spec((B, 8), row), spec((B, 8), row)]
    return pl.pallas_call(
        _tc_mlp_body,
        grid=grid,
        in_specs=in_specs,
        out_specs=out_specs,
        out_shape=[jax.ShapeDtypeStruct((NP, 8), jnp.float32),
                   jax.ShapeDtypeStruct((NP, 8), jnp.float32)],
    )(hl, hc, aeplus, hP, *weights)


def _sc_aggregate(px2d, sm2d, sd2d, rows2d, cols1d, zeros_hbm, NP, EP):
    ep_w = EP // NW
    KR = ep_w // 128
    stripe = NP // NS
    mesh = plsc.VectorSubcoreMesh(core_axis_name="c", subcore_axis_name="s",
                                  num_cores=NC, num_subcores=NS)

    @functools.partial(
        pl.kernel, mesh=mesh,
        compiler_params=pltpu.CompilerParams(use_tc_tiling_on_sc=False,
                                             needs_layout_passes=False),
        out_type=jax.ShapeDtypeStruct((NC, NP, D_AGG), jnp.float32),
        scratch_types=[pltpu.VMEM((KR, 128), jnp.int32),
                       pltpu.VMEM((ep_w,), jnp.int32),
                       pltpu.VMEM((NP // 128, 128), jnp.float32),
                       pltpu.VMEM((NP // 128, 128), jnp.float32),
                       pltpu.VMEM((NP // 128, 128), jnp.float32),
                       pltpu.VMEM((ep_w, D_AGG), jnp.float32),
                       pltpu.VMEM_SHARED((NP, D_AGG), jnp.float32),
                       pltpu.SemaphoreType.DMA],
    )
    def k(px_hbm, sm_hbm, sd_hbm, r_hbm, c_hbm, z_hbm, out_hbm, idxr, idxc,
          tvs, tvm, tvd, gbuf, acc, sem):
        s = lax.axis_index("s")
        c = lax.axis_index("c")
        wid = s * NC + c
        pltpu.sync_copy(z_hbm.at[pl.ds(s * stripe, stripe)],
                        acc.at[pl.ds(s * stripe, stripe)])
        pltpu.sync_copy(px_hbm, tvs)
        pltpu.sync_copy(sm_hbm, tvm)
        pltpu.sync_copy(sd_hbm, tvd)
        pltpu.sync_copy(c_hbm.at[pl.ds(wid * ep_w, ep_w)], idxc)
        pltpu.sync_copy(r_hbm.at[pl.ds(wid * KR, KR)], idxr)
        lane = lax.iota(jnp.int32, 16)
        zero16 = jnp.zeros((16,), jnp.int32)
        one16 = jnp.full((16,), 1, jnp.int32)
        two16 = jnp.full((16,), 2, jnp.int32)
        ones_f = jnp.full((16,), 1.0, jnp.float32)

        def body(i, carry):
            ci = idxc[pl.ds(i * 16, 16)]
            tr = lax.shift_right_logical(ci, 7)
            tc_ = jnp.bitwise_and(ci, 127)
            sv = (plsc.load_gather(tvs, [tr, tc_])
                  * plsc.load_gather(tvd, [tr, tc_]))
            sm = plsc.load_gather(tvm, [tr, tc_])
            e16 = lane + i * 16
            plsc.store_scatter(gbuf, [e16, zero16], sv)
            plsc.store_scatter(gbuf, [e16, one16], sm)
            plsc.store_scatter(gbuf, [e16, two16], ones_f)
            return carry

        lax.fori_loop(0, ep_w // 16, body, 0)
        plsc.subcore_barrier()
        cps = [pltpu.async_copy(gbuf.at[pl.ds(j * 128, 128)],
                                acc.at[idxr.at[j]], sem, add=True)
               for j in range(KR)]
        for cp in cps:
            cp.wait()
        plsc.subcore_barrier()
        pltpu.sync_copy(acc.at[pl.ds(s * stripe, stripe)],
                        out_hbm.at[c].at[pl.ds(s * stripe, stripe)])

    return k(px2d, sm2d, sd2d, rows2d, cols1d, zeros_hbm)


def _tc_final_body(part_ref, h_ref, vel_ref, x_ref, pv_ref,
                   a_ref, u_ref, b1_ref, w2_ref, b2_ref,
                   hn_ref, xn_ref, vo_ref):
    acc = part_ref[0] + part_ref[1]
    summed = acc[:, 0:1]
    smi = acc[:, 1:2]
    cnt = acc[:, 2:3]
    media = jnp.where(cnt > 0, summed / cnt, 0.0)
    pv0 = pv_ref[...][:, 0:1]
    velb = vel_ref[...] * pv0 + media
    xn_ref[...] = x_ref[...] + velb
    vo_ref[...] = velb
    hmid = jnp.tanh(jnp.dot(h_ref[...], a_ref[...],
                            preferred_element_type=jnp.float32)
                    + smi * u_ref[...] + b1_ref[...])
    hn_ref[...] = (jnp.dot(hmid, w2_ref[...],
                           preferred_element_type=jnp.float32) + b2_ref[...])


def _tc_final(part, hP, vel8, x8, pv, weights, NP, N_OUT, B=1024):
    grid = (NP // B,)
    row = lambda i: (i, 0)
    fix = lambda i: (0, 0)
    in_specs = [pl.BlockSpec((NC, B, D_AGG), lambda i: (0, i, 0)),
                pl.BlockSpec((B, 128), row), pl.BlockSpec((B, 8), row),
                pl.BlockSpec((B, 8), row), pl.BlockSpec((B, 8), row),
                pl.BlockSpec((128, 128), fix), pl.BlockSpec((1, 128), fix),
                pl.BlockSpec((1, 128), fix), pl.BlockSpec((128, 128), fix),
                pl.BlockSpec((1, 128), fix)]
    out_specs = [pl.BlockSpec((B, 128), row), pl.BlockSpec((B, 8), row),
                 pl.BlockSpec((B, 8), row)]
    return pl.pallas_call(
        _tc_final_body,
        grid=grid,
        in_specs=in_specs,
        out_specs=out_specs,
        out_shape=[jax.ShapeDtypeStruct((N_OUT, 128), jnp.float32),
                   jax.ShapeDtypeStruct((N_OUT, 8), jnp.float32),
                   jax.ShapeDtypeStruct((N_OUT, 8), jnp.float32)],
    )(part, hP, vel8, x8, pv, *weights)


def kernel(h, x, arestas, velocidade, atributos_arestas,
           pxw1, pxb1, pxw2, pxb2,
           pew1, peb1, pew2, peb2,
           phw1, phb1, phw2, phb2,
           pvw1, pvb1, pvw2, pvb2):
    f32 = jnp.float32
    N, ENT = h.shape
    E = arestas.shape[1]
    NP = _round_up(N, NW * 64)
    EP = _round_up(E, NW * 128)

    rows = arestas[0].astype(jnp.int32)
    cols = arestas[1].astype(jnp.int32)

    xt = jnp.pad(x.astype(f32), ((0, 0), (0, 1))).reshape(N * 4)
    rN = jnp.pad(rows[:N], (0, NP - N))
    cN = jnp.pad(cols[:N], (0, NP - N))
    hl, hc, rad1d, sd1d = _sc_gather(h.astype(f32), xt, rN, cN, NP)

    aeP = jnp.pad(atributos_arestas[:N].astype(f32), ((0, NP - N), (0, 0)))
    aeplus = jnp.concatenate(
        [aeP, rad1d[:, None], jnp.zeros((NP, 7), f32)], axis=1)
    we_aug = jnp.concatenate(
        [pew1[:, 2 * ENT + 1:].T, pew1[:, 2 * ENT].reshape(1, 128),
         jnp.zeros((7, 128), f32)], axis=0)
    hP = jnp.pad(h.astype(f32), ((0, NP - N), (0, 0)))
    mlp_weights = (
        pew1[:, :ENT].T, pew1[:, ENT:2 * ENT].T,
        we_aug,
        peb1.reshape(1, 128),
        pew2.T, peb2.reshape(1, 128),
        pxw1.T, pxb1.reshape(1, 128), pxw2.reshape(1, 128),
        pxb2.reshape(1, 1),
        pvw1.T, pvb1.reshape(1, 128), pvw2.reshape(1, 128),
        pvb2.reshape(1, 1),
    )
    T, pv = _tc_mlp(hl, hc, aeplus, hP, mlp_weights, NP)
    px2d = T[:, 0].reshape(NP // 128, 128)
    sm2d = T[:, 1].reshape(NP // 128, 128)
    sd2d = sd1d.reshape(NP // 128, 128)

    rowsP = jnp.concatenate(
        [rows, jnp.full((EP - E,), NP - 1, jnp.int32)]).reshape(EP // 128, 128)
    colsP = jnp.pad(cols, (0, EP - E))
    zer = jnp.zeros((NP, D_AGG), f32)
    part = _sc_aggregate(px2d, sm2d, sd2d, rowsP, colsP, zer, NP, EP)

    vel8 = jnp.pad(velocidade.astype(f32), ((0, NP - N), (0, 8 - 3)))
    x8 = jnp.pad(x.astype(f32), ((0, NP - N), (0, 8 - 3)))
    fin_weights = (phw1[:, :ENT].T, phw1[:, ENT].reshape(1, 128),
                   phb1.reshape(1, 128), phw2.T, phb2.reshape(1, 128))
    hn, xn8, vo8 = _tc_final(part, hP, vel8, x8, pv, fin_weights, NP, N)

    return (hn, xn8[:, :3], vo8[:, :3])
